# baseline (speedup 1.0000x reference)
import functools

import jax
import jax.numpy as jnp
import numpy as np
from jax.experimental import pallas as pl

N = 50000
E = 800000
IN_DIM = 128
EM_DIM = 64
ATTR_IN = 16
ATTR_EM = 16
NB = 10
RN = 100
MAX_RADIUS = 5.0
NUM_NEIGHBORS = 16.0
NUM_NODES = 50000.0

NODE_BLK = 1024
N_PAD = ((N + NODE_BLK - 1) // NODE_BLK) * NODE_BLK


def _fctp_body(x_ref, a_ref, w_ref, o_ref, *, scale):
    x = x_ref[...]
    a = a_ref[...]
    z = (x[:, :, None] * a[:, None, :]).reshape(x.shape[0], -1)
    o_ref[...] = jnp.dot(z, w_ref[...], preferred_element_type=jnp.float32) * scale


def _fctp(x, a, w):
    n, din = x.shape
    dj = a.shape[1]
    dout = w.shape[2]
    wf = w.reshape(din * dj, dout)
    scale = 1.0 / np.sqrt(din * dj)
    grid = (n // NODE_BLK,)
    return pl.pallas_call(
        functools.partial(_fctp_body, scale=scale),
        grid=grid,
        in_specs=[
            pl.BlockSpec((NODE_BLK, din), lambda i: (i, 0)),
            pl.BlockSpec((NODE_BLK, dj), lambda i: (i, 0)),
            pl.BlockSpec((din * dj, dout), lambda i: (0, 0)),
        ],
        out_specs=pl.BlockSpec((NODE_BLK, dout), lambda i: (i, 0)),
        out_shape=jax.ShapeDtypeStruct((n, dout), jnp.float32),
    )(x, a, wf)


def _smooth_cutoff(x):
    u = 2.0 * (x - 1.0)
    y = (1.0 - jnp.cos(np.pi * u)) / 2.0
    y = jnp.where(u > 0, 0.0, y)
    y = jnp.where(u < -1, 1.0, y)
    return y


def _radial_embed(el):
    centers = jnp.linspace(0.0, MAX_RADIUS, NB)
    step = MAX_RADIUS / (NB - 1)
    y = jnp.exp(-((el[:, None] - centers[None, :]) / step) ** 2) * 1.12
    return y * np.sqrt(NB)


def _conv(x, a, edge_src, edge_dst, edge_attr, edge_emb, Wsc, Wlin1, Wfc1, Wfc2, Wlin2):
    s = _fctp(x, a, Wsc)
    xl = _fctp(x, a, Wlin1)
    h = jax.nn.silu(edge_emb @ Wfc1 / np.sqrt(Wfc1.shape[0]))
    w = h @ Wfc2 / np.sqrt(Wfc2.shape[0])
    ef = xl[:N][edge_src] * edge_attr * w
    agg = jnp.zeros((N, xl.shape[1]), jnp.float32).at[edge_dst].add(ef) / np.sqrt(NUM_NEIGHBORS)
    agg = jnp.pad(agg, ((0, N_PAD - N), (0, 0)))
    out = _fctp(agg, a, Wlin2)
    c_s, c_x = np.sin(np.pi / 8.0), np.cos(np.pi / 8.0)
    return c_s * s + c_x * out


def kernel(x, x_attr, pos, edge_src, edge_dst, batch, W_em, b_em, W_ema, b_ema,
           sc0, lin1_0, fc1_0, fc2_0, lin2_0,
           sc1, lin1_1, fc1_1, fc2_1, lin2_1,
           sc2, lin1_2, fc1_2, fc2_2, lin2_2):
    h = x @ W_em + b_em
    a = x_attr @ W_ema + b_ema
    h = jnp.pad(h, ((0, N_PAD - N), (0, 0)))
    a = jnp.pad(a, ((0, N_PAD - N), (0, 0)))
    ev = pos[edge_src] - pos[edge_dst]
    el = jnp.sqrt(jnp.sum(ev * ev, axis=1) + 1e-12)
    emb = _radial_embed(el)
    edge_attr = _smooth_cutoff(el / MAX_RADIUS)[:, None]
    h = jax.nn.silu(_conv(h, a, edge_src, edge_dst, edge_attr, emb, sc0, lin1_0, fc1_0, fc2_0, lin2_0))
    h = jax.nn.silu(_conv(h, a, edge_src, edge_dst, edge_attr, emb, sc1, lin1_1, fc1_1, fc2_1, lin2_1))
    h = _conv(h, a, edge_src, edge_dst, edge_attr, emb, sc2, lin1_2, fc1_2, fc2_2, lin2_2)
    out = jnp.sum(h[:N], axis=0, keepdims=True) / np.sqrt(NUM_NODES)
    return out


# speedup vs baseline: 1.1590x; 1.1590x over previous
import functools

import jax
import jax.numpy as jnp
import numpy as np
from jax import lax
from jax.experimental import pallas as pl
from jax.experimental.pallas import tpu as pltpu
from jax.experimental.pallas import tpu_sc as plsc

N = 50000
E = 800000
IN_DIM = 128
EM_DIM = 64
ATTR_IN = 16
ATTR_EM = 16
NB = 10
RN = 100
MAX_RADIUS = 5.0
NUM_NEIGHBORS = 16.0
NUM_NODES = 50000.0

NODE_BLK = 1000
N_PAD = N

EP = 819200
CHUNK = 256
KROW = CHUNK // 128
TCHUNKS = EP // 16 // CHUNK
NROW = EP // 128
NA = 50048
ZROWS = NA // 16
ZQ = ZROWS // CHUNK
ZTAIL = ZROWS - ZQ * CHUNK


def _fctp_body(x_ref, a_ref, w_ref, o_ref, *, scale):
    x = x_ref[...]
    a = a_ref[...]
    z = (x[:, :, None] * a[:, None, :]).reshape(x.shape[0], -1)
    o_ref[...] = jnp.dot(z, w_ref[...], preferred_element_type=jnp.float32) * scale


def _fctp(x, a, w):
    n, din = x.shape
    dj = a.shape[1]
    dout = w.shape[2]
    wf = w.reshape(din * dj, dout)
    scale = 1.0 / np.sqrt(din * dj)
    grid = (n // NODE_BLK,)
    return pl.pallas_call(
        functools.partial(_fctp_body, scale=scale),
        grid=grid,
        in_specs=[
            pl.BlockSpec((NODE_BLK, din), lambda i: (i, 0)),
            pl.BlockSpec((NODE_BLK, dj), lambda i: (i, 0)),
            pl.BlockSpec((din * dj, dout), lambda i: (0, 0)),
        ],
        out_specs=pl.BlockSpec((NODE_BLK, dout), lambda i: (i, 0)),
        out_shape=jax.ShapeDtypeStruct((n, dout), jnp.float32),
    )(x, a, wf)


def _gs_body(xl_hbm, cw_hbm, srcp_hbm, dst_hbm, out_hbm,
             idx_s, idx_d, rows, cwv, aggsh, sem):
    c = lax.axis_index("c")
    s = lax.axis_index("s")

    zero16 = jnp.zeros((16,), jnp.float32)

    def zb_body(r, carry):
        rows[r, pl.ds(0, 16)] = zero16
        rows[r, pl.ds(16, 16)] = zero16
        return carry

    lax.fori_loop(0, CHUNK, zb_body, 0)
    for q in range(ZQ):
        pltpu.sync_copy(rows, aggsh.at[pl.ds(s * ZROWS + q * CHUNK, CHUNK)])
    pltpu.sync_copy(rows.at[pl.ds(0, ZTAIL)],
                    aggsh.at[pl.ds(s * ZROWS + ZQ * CHUNK, ZTAIL)])
    plsc.subcore_barrier()

    base_row = s * (TCHUNKS * KROW)

    def chunk_body(g, carry):
        j0 = base_row + g * KROW
        e0 = j0 * 128
        pltpu.sync_copy(srcp_hbm.at[pl.ds(c * NROW + j0, KROW)], idx_s)
        pltpu.sync_copy(dst_hbm.at[pl.ds(j0, KROW)], idx_d)
        pltpu.sync_copy(cw_hbm.at[pl.ds(c * EP + e0, CHUNK)], cwv)
        descs = [
            pltpu.async_copy(xl_hbm.at[idx_s.at[j]],
                             rows.at[pl.ds(j * 128, 128)], sem)
            for j in range(KROW)
        ]
        for d in descs:
            d.wait()

        def mul_body(r, cc):
            rows[r, pl.ds(0, 16)] = rows[r, pl.ds(0, 16)] * cwv[r, pl.ds(0, 16)]
            rows[r, pl.ds(16, 16)] = rows[r, pl.ds(16, 16)] * cwv[r, pl.ds(16, 16)]
            return cc

        lax.fori_loop(0, CHUNK, mul_body, 0)
        for j in range(KROW):
            pltpu.sync_copy(rows.at[pl.ds(j * 128, 128)],
                            aggsh.at[idx_d.at[j]], add=True)
        return carry

    lax.fori_loop(0, TCHUNKS, chunk_body, 0)
    plsc.subcore_barrier()
    pltpu.sync_copy(aggsh.at[pl.ds(s * ZROWS, ZROWS)],
                    out_hbm.at[pl.ds(c * NA + s * ZROWS, ZROWS)])


_gs = functools.partial(
    pl.kernel,
    mesh=plsc.VectorSubcoreMesh(core_axis_name="c", subcore_axis_name="s"),
    compiler_params=pltpu.CompilerParams(use_tc_tiling_on_sc=False),
    out_type=jax.ShapeDtypeStruct((2 * NA, 32), jnp.float32),
    scratch_types=[
        pltpu.VMEM((KROW, 128), jnp.int32),
        pltpu.VMEM((KROW, 128), jnp.int32),
        pltpu.VMEM((CHUNK, 32), jnp.float32),
        pltpu.VMEM((CHUNK, 32), jnp.float32),
        pltpu.VMEM_SHARED((NA, 32), jnp.float32),
        pltpu.SemaphoreType.DMA,
    ],
)(_gs_body)


def _smooth_cutoff(x):
    u = 2.0 * (x - 1.0)
    y = (1.0 - jnp.cos(np.pi * u)) / 2.0
    y = jnp.where(u > 0, 0.0, y)
    y = jnp.where(u < -1, 1.0, y)
    return y


def _radial_embed(el):
    centers = jnp.linspace(0.0, MAX_RADIUS, NB)
    step = MAX_RADIUS / (NB - 1)
    y = jnp.exp(-((el[:, None] - centers[None, :]) / step) ** 2) * 1.12
    return y * np.sqrt(NB)


def _conv(x, a, srcp2d, dst2d, edge_attr, edge_emb, Wsc, Wlin1, Wfc1, Wfc2, Wlin2):
    s = _fctp(x, a, Wsc)
    xl = _fctp(x, a, Wlin1)
    h = jax.nn.silu(edge_emb @ Wfc1 / np.sqrt(Wfc1.shape[0]))
    w = h @ Wfc2 / np.sqrt(Wfc2.shape[0])
    cw = (edge_attr * w) * (1.0 / np.sqrt(NUM_NEIGHBORS))
    cw = jnp.pad(cw, ((0, EP - E), (0, 0)))
    cwf = cw.reshape(EP, 2, 32).transpose(1, 0, 2).reshape(2 * EP, 32)
    xl2n = xl[:N].reshape(N, 2, 32).transpose(1, 0, 2).reshape(2 * N, 32)
    agg2n = _gs(xl2n, cwf, srcp2d, dst2d)
    agg = agg2n.reshape(2, NA, 32)[:, :N].transpose(1, 0, 2).reshape(N, 64)
    out = _fctp(agg, a, Wlin2)
    c_s, c_x = np.sin(np.pi / 8.0), np.cos(np.pi / 8.0)
    return c_s * s + c_x * out


def kernel(x, x_attr, pos, edge_src, edge_dst, batch, W_em, b_em, W_ema, b_ema,
           sc0, lin1_0, fc1_0, fc2_0, lin2_0,
           sc1, lin1_1, fc1_1, fc2_1, lin2_1,
           sc2, lin1_2, fc1_2, fc2_2, lin2_2):
    h = x @ W_em + b_em
    a = x_attr @ W_ema + b_ema
    ev = pos[edge_src] - pos[edge_dst]
    el = jnp.sqrt(jnp.sum(ev * ev, axis=1) + 1e-12)
    emb = _radial_embed(el)
    edge_attr = _smooth_cutoff(el / MAX_RADIUS)[:, None]

    srcpad = jnp.pad(edge_src, (0, EP - E))
    srcp2d = jnp.concatenate([srcpad, srcpad + N]).reshape(2 * NROW, 128)
    dst2d = jnp.pad(edge_dst, (0, EP - E)).reshape(NROW, 128)

    h = jax.nn.silu(_conv(h, a, srcp2d, dst2d, edge_attr, emb, sc0, lin1_0, fc1_0, fc2_0, lin2_0))
    h = jax.nn.silu(_conv(h, a, srcp2d, dst2d, edge_attr, emb, sc1, lin1_1, fc1_1, fc2_1, lin2_1))
    h = _conv(h, a, srcp2d, dst2d, edge_attr, emb, sc2, lin1_2, fc1_2, fc2_2, lin2_2)
    out = jnp.sum(h[:N], axis=0, keepdims=True) / np.sqrt(NUM_NODES)
    return out


# speedup vs baseline: 1.4500x; 1.2511x over previous
import functools

import jax
import jax.numpy as jnp
import numpy as np
from jax import lax
from jax.experimental import pallas as pl
from jax.experimental.pallas import tpu as pltpu
from jax.experimental.pallas import tpu_sc as plsc

N = 50000
E = 800000
NB = 10
RN = 100
MAX_RADIUS = 5.0

NA = 50048
NBLK = 1472
NGRID = NA // NBLK

EP = 819200
CHUNK = 256
KROW = CHUNK // 128
NROW = EP // 128
TCHUNKS = EP // 16 // CHUNK
GCHUNKS = EP // 32 // CHUNK
ZROWS = NA // 16
ZQ = ZROWS // CHUNK
ZTAIL = ZROWS - ZQ * CHUNK

EBLK = 2048
C_S = float(np.sin(np.pi / 8.0))
C_X = float(np.cos(np.pi / 8.0))
FSCALE = 1.0 / 32.0

_SC_PARAMS = pltpu.CompilerParams(use_tc_tiling_on_sc=False,
                                  needs_layout_passes=False)
_MESH = plsc.VectorSubcoreMesh(core_axis_name="c", subcore_axis_name="s")


def _geom_body(pos_hbm, src_hbm, dst_hbm, out_hbm, idx_s, idx_d, ps, pd, sbuf, sem):
    c = lax.axis_index("c")
    s = lax.axis_index("s")
    wid = s * 2 + c
    base_row = wid * (GCHUNKS * KROW)
    lanes = lax.iota(jnp.int32, 16)
    col0 = jnp.zeros((16,), jnp.int32)
    col1 = col0 + 1
    col2 = col0 + 2

    def chunk_body(g, carry):
        j0 = base_row + g * KROW
        pltpu.sync_copy(src_hbm.at[pl.ds(j0, KROW)], idx_s)
        pltpu.sync_copy(dst_hbm.at[pl.ds(j0, KROW)], idx_d)
        ds_ = pltpu.async_copy(pos_hbm.at[idx_s.at[0]], ps.at[pl.ds(0, 128)], sem)
        ds2 = pltpu.async_copy(pos_hbm.at[idx_s.at[1]], ps.at[pl.ds(128, 128)], sem)
        dd_ = pltpu.async_copy(pos_hbm.at[idx_d.at[0]], pd.at[pl.ds(0, 128)], sem)
        dd2 = pltpu.async_copy(pos_hbm.at[idx_d.at[1]], pd.at[pl.ds(128, 128)], sem)
        ds_.wait(); ds2.wait(); dd_.wait(); dd2.wait()
        for v in range(16):
            ri = lanes + (v * 16)
            dx = plsc.load_gather(ps, [ri, col0]) - plsc.load_gather(pd, [ri, col0])
            dy = plsc.load_gather(ps, [ri, col1]) - plsc.load_gather(pd, [ri, col1])
            dz = plsc.load_gather(ps, [ri, col2]) - plsc.load_gather(pd, [ri, col2])
            sbuf[pl.ds(v * 16, 16)] = dx * dx + dy * dy + dz * dz
        pltpu.sync_copy(sbuf, out_hbm.at[pl.ds(j0 * 128, CHUNK)])
        return carry

    lax.fori_loop(0, GCHUNKS, chunk_body, 0)


_geom = functools.partial(
    pl.kernel,
    mesh=_MESH,
    compiler_params=_SC_PARAMS,
    out_type=jax.ShapeDtypeStruct((EP,), jnp.float32),
    scratch_types=[
        pltpu.VMEM((KROW, 128), jnp.int32),
        pltpu.VMEM((KROW, 128), jnp.int32),
        pltpu.VMEM((CHUNK, 16), jnp.float32),
        pltpu.VMEM((CHUNK, 16), jnp.float32),
        pltpu.VMEM((CHUNK,), jnp.float32),
        pltpu.SemaphoreType.DMA,
    ],
)(_geom_body)


def _make_gs(layer):
    def _gs_body(xl_hbm, cw_hbm, srcp_hbm, dst_hbm, out_hbm,
                 idx_s, idx_d, rows, cwv, aggsh, sem):
        c = lax.axis_index("c")
        s = lax.axis_index("s")
        zero16 = jnp.zeros((16,), jnp.float32)

        def zb_body(r, carry):
            rows[r, pl.ds(0, 16)] = zero16
            rows[r, pl.ds(16, 16)] = zero16
            return carry

        lax.fori_loop(0, CHUNK, zb_body, 0)
        for q in range(ZQ):
            pltpu.sync_copy(rows, aggsh.at[pl.ds(s * ZROWS + q * CHUNK, CHUNK)])
        pltpu.sync_copy(rows.at[pl.ds(0, ZTAIL)],
                        aggsh.at[pl.ds(s * ZROWS + ZQ * CHUNK, ZTAIL)])
        plsc.subcore_barrier()

        base_row = s * (TCHUNKS * KROW)
        cw_base = (2 * layer + c) * EP

        def chunk_body(g, carry):
            j0 = base_row + g * KROW
            e0 = j0 * 128
            pltpu.sync_copy(srcp_hbm.at[pl.ds(c * NROW + j0, KROW)], idx_s)
            pltpu.sync_copy(dst_hbm.at[pl.ds(j0, KROW)], idx_d)
            pltpu.sync_copy(cw_hbm.at[pl.ds(cw_base + e0, CHUNK)], cwv)
            descs = [
                pltpu.async_copy(xl_hbm.at[idx_s.at[j]],
                                 rows.at[pl.ds(j * 128, 128)], sem)
                for j in range(KROW)
            ]
            for d in descs:
                d.wait()

            def mul_body(r, cc):
                rows[r, pl.ds(0, 16)] = rows[r, pl.ds(0, 16)] * cwv[r, pl.ds(0, 16)]
                rows[r, pl.ds(16, 16)] = rows[r, pl.ds(16, 16)] * cwv[r, pl.ds(16, 16)]
                return cc

            lax.fori_loop(0, CHUNK, mul_body, 0)
            for j in range(KROW):
                pltpu.sync_copy(rows.at[pl.ds(j * 128, 128)],
                                aggsh.at[idx_d.at[j]], add=True)
            return carry

        lax.fori_loop(0, TCHUNKS, chunk_body, 0)
        plsc.subcore_barrier()
        pltpu.sync_copy(aggsh.at[pl.ds(s * ZROWS, ZROWS)],
                        out_hbm.at[pl.ds(c * NA + s * ZROWS, ZROWS)])

    return functools.partial(
        pl.kernel,
        mesh=_MESH,
        compiler_params=_SC_PARAMS,
        out_type=jax.ShapeDtypeStruct((2 * NA, 32), jnp.float32),
        scratch_types=[
            pltpu.VMEM((KROW, 128), jnp.int32),
            pltpu.VMEM((KROW, 128), jnp.int32),
            pltpu.VMEM((CHUNK, 32), jnp.float32),
            pltpu.VMEM((CHUNK, 32), jnp.float32),
            pltpu.VMEM_SHARED((NA, 32), jnp.float32),
            pltpu.SemaphoreType.DMA,
        ],
    )(_gs_body)


_gs = [_make_gs(l) for l in range(3)]


def _k1_body(x_ref, xa_ref, wem_ref, bem_ref, wema_ref, bema_ref, wcat_ref,
             a_ref, s_ref, xl_ref):
    h = jnp.dot(x_ref[...], wem_ref[...], preferred_element_type=jnp.float32) + bem_ref[...]
    a = jnp.dot(xa_ref[...], wema_ref[...], preferred_element_type=jnp.float32) + bema_ref[...]
    a_ref[...] = a
    z = (h[:, :, None] * a[:, None, :]).reshape(NBLK, 1024)
    szl = jnp.dot(z, wcat_ref[...], preferred_element_type=jnp.float32) * FSCALE
    s_ref[...] = szl[:, :64]
    xl_ref[0] = szl[:, 64:96]
    xl_ref[1] = szl[:, 96:128]


def _k1(xp, xap, W_em, b_em, W_ema, b_ema, wcat):
    return pl.pallas_call(
        _k1_body,
        grid=(NGRID,),
        in_specs=[
            pl.BlockSpec((NBLK, 128), lambda i: (i, 0)),
            pl.BlockSpec((NBLK, 16), lambda i: (i, 0)),
            pl.BlockSpec((128, 64), lambda i: (0, 0)),
            pl.BlockSpec((1, 64), lambda i: (0, 0)),
            pl.BlockSpec((16, 16), lambda i: (0, 0)),
            pl.BlockSpec((1, 16), lambda i: (0, 0)),
            pl.BlockSpec((1024, 128), lambda i: (0, 0)),
        ],
        out_specs=[
            pl.BlockSpec((NBLK, 16), lambda i: (i, 0)),
            pl.BlockSpec((NBLK, 64), lambda i: (i, 0)),
            pl.BlockSpec((2, NBLK, 32), lambda i: (0, i, 0)),
        ],
        out_shape=[
            jax.ShapeDtypeStruct((NA, 16), jnp.float32),
            jax.ShapeDtypeStruct((NA, 64), jnp.float32),
            jax.ShapeDtypeStruct((2, NA, 32), jnp.float32),
        ],
    )(xp, xap, W_em, b_em, W_ema, b_ema, wcat)


def _kedge_body(se_ref, fc1_ref, fc2_ref, cw_ref):
    i = pl.program_id(0)
    s = se_ref[...]
    el = jnp.sqrt(s + 1e-12)
    u = 2.0 * (el * (1.0 / MAX_RADIUS) - 1.0)
    y = (1.0 - jnp.cos(np.pi * u)) * 0.5
    y = jnp.where(u > 0, 0.0, y)
    y = jnp.where(u < -1, 1.0, y)
    rid = i * EBLK + lax.broadcasted_iota(jnp.int32, (EBLK, 1), 0)
    y = jnp.where(rid < E, y, 0.0) * 0.25
    step = MAX_RADIUS / (NB - 1)
    centers = lax.broadcasted_iota(jnp.int32, (1, NB), 1).astype(jnp.float32) * step
    emb = jnp.exp(-((el - centers) * (1.0 / step)) ** 2) * (1.12 * np.sqrt(NB))
    for l in range(3):
        g = jnp.dot(emb, fc1_ref[l], preferred_element_type=jnp.float32) * (1.0 / np.sqrt(NB))
        h = g * jax.nn.sigmoid(g)
        w = jnp.dot(h, fc2_ref[l], preferred_element_type=jnp.float32) * 0.1
        cw = y * w
        cw_ref[l, 0] = cw[:, :32]
        cw_ref[l, 1] = cw[:, 32:]


def _kedge(se2, fc1_all, fc2_all):
    return pl.pallas_call(
        _kedge_body,
        grid=(EP // EBLK,),
        in_specs=[
            pl.BlockSpec((EBLK, 1), lambda i: (i, 0)),
            pl.BlockSpec((3, 10, 100), lambda i: (0, 0, 0)),
            pl.BlockSpec((3, 100, 64), lambda i: (0, 0, 0)),
        ],
        out_specs=pl.BlockSpec((3, 2, EBLK, 32), lambda i: (0, 0, i, 0)),
        out_shape=jax.ShapeDtypeStruct((3, 2, EP, 32), jnp.float32),
    )(se2, fc1_all, fc2_all)


def _kmid_body(agg_ref, a_ref, s_ref, wlin2_ref, wcat_ref, s_next_ref, xl_next_ref):
    a = a_ref[...]
    agg = jnp.concatenate([agg_ref[0], agg_ref[1]], axis=1)
    z = (agg[:, :, None] * a[:, None, :]).reshape(NBLK, 1024)
    o = jnp.dot(z, wlin2_ref[...], preferred_element_type=jnp.float32) * FSCALE
    hcur = C_S * s_ref[...] + C_X * o
    hcur = hcur * jax.nn.sigmoid(hcur)
    z2 = (hcur[:, :, None] * a[:, None, :]).reshape(NBLK, 1024)
    szl = jnp.dot(z2, wcat_ref[...], preferred_element_type=jnp.float32) * FSCALE
    s_next_ref[...] = szl[:, :64]
    xl_next_ref[0] = szl[:, 64:96]
    xl_next_ref[1] = szl[:, 96:128]


def _kmid(agg, a, s_cur, wlin2f, wcat):
    return pl.pallas_call(
        _kmid_body,
        grid=(NGRID,),
        in_specs=[
            pl.BlockSpec((2, NBLK, 32), lambda i: (0, i, 0)),
            pl.BlockSpec((NBLK, 16), lambda i: (i, 0)),
            pl.BlockSpec((NBLK, 64), lambda i: (i, 0)),
            pl.BlockSpec((1024, 64), lambda i: (0, 0)),
            pl.BlockSpec((1024, 128), lambda i: (0, 0)),
        ],
        out_specs=[
            pl.BlockSpec((NBLK, 64), lambda i: (i, 0)),
            pl.BlockSpec((2, NBLK, 32), lambda i: (0, i, 0)),
        ],
        out_shape=[
            jax.ShapeDtypeStruct((NA, 64), jnp.float32),
            jax.ShapeDtypeStruct((2, NA, 32), jnp.float32),
        ],
    )(agg, a, s_cur, wlin2f, wcat)


def _kfin_body(agg_ref, a_ref, s_ref, wlin2_ref, o_ref):
    i = pl.program_id(0)
    a = a_ref[...]
    agg = jnp.concatenate([agg_ref[0], agg_ref[1]], axis=1)
    z = (agg[:, :, None] * a[:, None, :]).reshape(NBLK, 1024)
    o = jnp.dot(z, wlin2_ref[...], preferred_element_type=jnp.float32) * FSCALE
    hout = C_S * s_ref[:, 0:1] + C_X * o
    rid = i * NBLK + lax.broadcasted_iota(jnp.int32, (NBLK, 1), 0)
    hout = jnp.where(rid < N, hout, 0.0)
    part = (jnp.sum(hout) * (1.0 / np.sqrt(float(N)))).reshape(1, 1)

    @pl.when(i == 0)
    def _():
        o_ref[...] = jnp.zeros((1, 1), jnp.float32)

    o_ref[...] += part


def _kfin(agg, a, s_cur, wlin2f):
    return pl.pallas_call(
        _kfin_body,
        grid=(NGRID,),
        in_specs=[
            pl.BlockSpec((2, NBLK, 32), lambda i: (0, i, 0)),
            pl.BlockSpec((NBLK, 16), lambda i: (i, 0)),
            pl.BlockSpec((NBLK, 64), lambda i: (i, 0)),
            pl.BlockSpec((1024, 1), lambda i: (0, 0)),
        ],
        out_specs=pl.BlockSpec((1, 1), lambda i: (0, 0)),
        out_shape=jax.ShapeDtypeStruct((1, 1), jnp.float32),
    )(agg, a, s_cur, wlin2f)


def kernel(x, x_attr, pos, edge_src, edge_dst, batch, W_em, b_em, W_ema, b_ema,
           sc0, lin1_0, fc1_0, fc2_0, lin2_0,
           sc1, lin1_1, fc1_1, fc2_1, lin2_1,
           sc2, lin1_2, fc1_2, fc2_2, lin2_2):
    xp = jnp.pad(x, ((0, NA - N), (0, 0)))
    xap = jnp.pad(x_attr, ((0, NA - N), (0, 0)))
    pos16 = jnp.pad(pos, ((0, 0), (0, 13)))

    srcpad = jnp.pad(edge_src, (0, EP - E))
    dstpad = jnp.pad(edge_dst, (0, EP - E))
    src2d = srcpad.reshape(NROW, 128)
    dst2d = dstpad.reshape(NROW, 128)
    srcp2d = jnp.concatenate([srcpad, srcpad + NA]).reshape(2 * NROW, 128)

    wcat0 = jnp.concatenate([sc0.reshape(1024, 64), lin1_0.reshape(1024, 64)], axis=1)
    wcat1 = jnp.concatenate([sc1.reshape(1024, 64), lin1_1.reshape(1024, 64)], axis=1)
    wcat2 = jnp.concatenate([jnp.pad(sc2.reshape(1024, 1), ((0, 0), (0, 63))),
                             lin1_2.reshape(1024, 64)], axis=1)
    fc1_all = jnp.stack([fc1_0, fc1_1, fc1_2])
    fc2_all = jnp.stack([fc2_0, fc2_1, fc2_2])

    s_e = _geom(pos16, src2d, dst2d)
    cw_all = _kedge(s_e.reshape(EP, 1), fc1_all, fc2_all)
    cw_flat = cw_all.reshape(6 * EP, 32)

    a, s0, xl0 = _k1(xp, xap, W_em, b_em[None, :], W_ema, b_ema[None, :], wcat0)

    agg0 = _gs[0](xl0.reshape(2 * NA, 32), cw_flat, srcp2d, dst2d).reshape(2, NA, 32)
    s1, xl1 = _kmid(agg0, a, s0, lin2_0.reshape(1024, 64), wcat1)

    agg1 = _gs[1](xl1.reshape(2 * NA, 32), cw_flat, srcp2d, dst2d).reshape(2, NA, 32)
    s2, xl2 = _kmid(agg1, a, s1, lin2_1.reshape(1024, 64), wcat2)

    agg2 = _gs[2](xl2.reshape(2 * NA, 32), cw_flat, srcp2d, dst2d).reshape(2, NA, 32)
    return _kfin(agg2, a, s2, lin2_2.reshape(1024, 1))


# speedup vs baseline: 2.0635x; 1.4230x over previous
import functools

import jax
import jax.numpy as jnp
import numpy as np
from jax import lax
from jax.experimental import pallas as pl
from jax.experimental.pallas import tpu as pltpu
from jax.experimental.pallas import tpu_sc as plsc

N = 50000
E = 800000
NB = 10
RN = 100
MAX_RADIUS = 5.0

NA = 50048
NBLK = 1472
NGRID = NA // NBLK

EP = 819200
CHUNK = 256
KROW = CHUNK // 128
NROW = EP // 128
TCHUNKS = EP // 16 // CHUNK
GCHUNKS = EP // 32 // CHUNK
ZROWS = NA // 16
ZQ = ZROWS // CHUNK
ZTAIL = ZROWS - ZQ * CHUNK

EBLK = 4096
C_S = float(np.sin(np.pi / 8.0))
C_X = float(np.cos(np.pi / 8.0))
FSCALE = 1.0 / 32.0

_SC_PARAMS = pltpu.CompilerParams(use_tc_tiling_on_sc=False,
                                  needs_layout_passes=False)
_MESH = plsc.VectorSubcoreMesh(core_axis_name="c", subcore_axis_name="s")


def _geom_body(pos_hbm, src_hbm, dst_hbm, out_hbm, idx_s, idx_d, ps, pd, sbuf, sem):
    c = lax.axis_index("c")
    s = lax.axis_index("s")
    wid = s * 2 + c
    base_row = wid * (GCHUNKS * KROW)
    lanes = lax.iota(jnp.int32, 16)
    col0 = jnp.zeros((16,), jnp.int32)
    col1 = col0 + 1
    col2 = col0 + 2

    def chunk_body(g, carry):
        j0 = base_row + g * KROW
        pltpu.sync_copy(src_hbm.at[pl.ds(j0, KROW)], idx_s)
        pltpu.sync_copy(dst_hbm.at[pl.ds(j0, KROW)], idx_d)
        ds_ = pltpu.async_copy(pos_hbm.at[idx_s.at[0]], ps.at[pl.ds(0, 128)], sem)
        ds2 = pltpu.async_copy(pos_hbm.at[idx_s.at[1]], ps.at[pl.ds(128, 128)], sem)
        dd_ = pltpu.async_copy(pos_hbm.at[idx_d.at[0]], pd.at[pl.ds(0, 128)], sem)
        dd2 = pltpu.async_copy(pos_hbm.at[idx_d.at[1]], pd.at[pl.ds(128, 128)], sem)
        ds_.wait(); ds2.wait(); dd_.wait(); dd2.wait()
        for v in range(16):
            ri = lanes + (v * 16)
            dx = plsc.load_gather(ps, [ri, col0]) - plsc.load_gather(pd, [ri, col0])
            dy = plsc.load_gather(ps, [ri, col1]) - plsc.load_gather(pd, [ri, col1])
            dz = plsc.load_gather(ps, [ri, col2]) - plsc.load_gather(pd, [ri, col2])
            sbuf[pl.ds(v * 16, 16)] = dx * dx + dy * dy + dz * dz
        pltpu.sync_copy(sbuf, out_hbm.at[pl.ds(j0 * 128, CHUNK)])
        return carry

    lax.fori_loop(0, GCHUNKS, chunk_body, 0)


_geom = functools.partial(
    pl.kernel,
    mesh=_MESH,
    compiler_params=_SC_PARAMS,
    out_type=jax.ShapeDtypeStruct((EP,), jnp.float32),
    scratch_types=[
        pltpu.VMEM((KROW, 128), jnp.int32),
        pltpu.VMEM((KROW, 128), jnp.int32),
        pltpu.VMEM((CHUNK, 16), jnp.float32),
        pltpu.VMEM((CHUNK, 16), jnp.float32),
        pltpu.VMEM((CHUNK,), jnp.float32),
        pltpu.SemaphoreType.DMA,
    ],
)(_geom_body)


def _make_gs(layer):
    def _gs_body(xl_hbm, cw_hbm, srcp_hbm, dst_hbm, out_hbm,
                 idx_s, idx_d, rows, cwv, aggsh, sem):
        c = lax.axis_index("c")
        s = lax.axis_index("s")
        zero16 = jnp.zeros((16,), jnp.float32)

        def zb_body(r, carry):
            rows[r, pl.ds(0, 16)] = zero16
            rows[r, pl.ds(16, 16)] = zero16
            return carry

        lax.fori_loop(0, CHUNK, zb_body, 0)
        for q in range(ZQ):
            pltpu.sync_copy(rows, aggsh.at[pl.ds(s * ZROWS + q * CHUNK, CHUNK)])
        pltpu.sync_copy(rows.at[pl.ds(0, ZTAIL)],
                        aggsh.at[pl.ds(s * ZROWS + ZQ * CHUNK, ZTAIL)])
        plsc.subcore_barrier()

        base_row = s * (TCHUNKS * KROW)
        cw_base = (2 * layer + c) * EP

        def chunk_body(g, carry):
            j0 = base_row + g * KROW
            e0 = j0 * 128
            pltpu.sync_copy(srcp_hbm.at[pl.ds(c * NROW + j0, KROW)], idx_s)
            pltpu.sync_copy(dst_hbm.at[pl.ds(j0, KROW)], idx_d)
            pltpu.sync_copy(cw_hbm.at[pl.ds(cw_base + e0, CHUNK)], cwv)
            descs = [
                pltpu.async_copy(xl_hbm.at[idx_s.at[j]],
                                 rows.at[pl.ds(j * 128, 128)], sem)
                for j in range(KROW)
            ]
            for d in descs:
                d.wait()

            def mul_body(r, cc):
                rows[r, pl.ds(0, 16)] = rows[r, pl.ds(0, 16)] * cwv[r, pl.ds(0, 16)]
                rows[r, pl.ds(16, 16)] = rows[r, pl.ds(16, 16)] * cwv[r, pl.ds(16, 16)]
                return cc

            lax.fori_loop(0, CHUNK, mul_body, 0)
            for j in range(KROW):
                pltpu.sync_copy(rows.at[pl.ds(j * 128, 128)],
                                aggsh.at[idx_d.at[j]], add=True)
            return carry

        lax.fori_loop(0, TCHUNKS, chunk_body, 0)
        plsc.subcore_barrier()
        pltpu.sync_copy(aggsh.at[pl.ds(s * ZROWS, ZROWS)],
                        out_hbm.at[pl.ds(c * NA + s * ZROWS, ZROWS)])

    return functools.partial(
        pl.kernel,
        mesh=_MESH,
        compiler_params=_SC_PARAMS,
        out_type=jax.ShapeDtypeStruct((2 * NA, 32), jnp.float32),
        scratch_types=[
            pltpu.VMEM((KROW, 128), jnp.int32),
            pltpu.VMEM((KROW, 128), jnp.int32),
            pltpu.VMEM((CHUNK, 32), jnp.float32),
            pltpu.VMEM((CHUNK, 32), jnp.float32),
            pltpu.VMEM_SHARED((NA, 32), jnp.float32),
            pltpu.SemaphoreType.DMA,
        ],
    )(_gs_body)


_gs = [_make_gs(l) for l in range(3)]


def _zcat(xmat, a):
    return jnp.concatenate([xmat * a[:, j:j + 1] for j in range(16)], axis=1)


def _k1_body(x_ref, xa_ref, wem_ref, bem_ref, wema_ref, bema_ref, wcat_ref,
             a_ref, s_ref, xl_ref):
    h = jnp.dot(x_ref[...], wem_ref[...], preferred_element_type=jnp.float32) + bem_ref[...]
    a = jnp.dot(xa_ref[...], wema_ref[...], preferred_element_type=jnp.float32) + bema_ref[...]
    a_ref[...] = a
    z = _zcat(h, a)
    szl = jnp.dot(z, wcat_ref[...], preferred_element_type=jnp.float32) * FSCALE
    s_ref[...] = szl[:, :64]
    xl_ref[0] = szl[:, 64:96]
    xl_ref[1] = szl[:, 96:128]


def _k1(xp, xap, W_em, b_em, W_ema, b_ema, wcat):
    return pl.pallas_call(
        _k1_body,
        grid=(NGRID,),
        in_specs=[
            pl.BlockSpec((NBLK, 128), lambda i: (i, 0)),
            pl.BlockSpec((NBLK, 16), lambda i: (i, 0)),
            pl.BlockSpec((128, 64), lambda i: (0, 0)),
            pl.BlockSpec((1, 64), lambda i: (0, 0)),
            pl.BlockSpec((16, 16), lambda i: (0, 0)),
            pl.BlockSpec((1, 16), lambda i: (0, 0)),
            pl.BlockSpec((1024, 128), lambda i: (0, 0)),
        ],
        out_specs=[
            pl.BlockSpec((NBLK, 16), lambda i: (i, 0)),
            pl.BlockSpec((NBLK, 64), lambda i: (i, 0)),
            pl.BlockSpec((2, NBLK, 32), lambda i: (0, i, 0)),
        ],
        out_shape=[
            jax.ShapeDtypeStruct((NA, 16), jnp.float32),
            jax.ShapeDtypeStruct((NA, 64), jnp.float32),
            jax.ShapeDtypeStruct((2, NA, 32), jnp.float32),
        ],
    )(xp, xap, W_em, b_em, W_ema, b_ema, wcat)


def _kedge_body(se_ref, fc1_ref, fc2_ref, cw_ref):
    i = pl.program_id(0)
    s = se_ref[...]
    el = jnp.sqrt(s + 1e-12)
    u = 2.0 * (el * (1.0 / MAX_RADIUS) - 1.0)
    y = (1.0 - jnp.cos(np.pi * u)) * 0.5
    y = jnp.where(u > 0, 0.0, y)
    y = jnp.where(u < -1, 1.0, y)
    rid = i * EBLK + lax.broadcasted_iota(jnp.int32, (EBLK, 1), 0)
    y = jnp.where(rid < E, y, 0.0) * 0.25
    step = MAX_RADIUS / (NB - 1)
    centers = lax.broadcasted_iota(jnp.int32, (1, 16), 1).astype(jnp.float32) * step
    emb = jnp.exp(-((el - centers) * (1.0 / step)) ** 2) * (1.12 * np.sqrt(NB))
    g = jnp.dot(emb, fc1_ref[...], preferred_element_type=jnp.float32) * (1.0 / np.sqrt(NB))
    h = g * jax.nn.sigmoid(g)
    w = jnp.dot(h, fc2_ref[...], preferred_element_type=jnp.float32) * 0.1
    cw = y * w
    for l in range(3):
        cw_ref[l, 0] = cw[:, 64 * l:64 * l + 32]
        cw_ref[l, 1] = cw[:, 64 * l + 32:64 * l + 64]


def _kedge(se2, fc1_cat, fc2_bd):
    return pl.pallas_call(
        _kedge_body,
        grid=(EP // EBLK,),
        in_specs=[
            pl.BlockSpec((EBLK, 1), lambda i: (i, 0)),
            pl.BlockSpec((16, 304), lambda i: (0, 0)),
            pl.BlockSpec((304, 192), lambda i: (0, 0)),
        ],
        out_specs=pl.BlockSpec((3, 2, EBLK, 32), lambda i: (0, 0, i, 0)),
        out_shape=jax.ShapeDtypeStruct((3, 2, EP, 32), jnp.float32),
    )(se2, fc1_cat, fc2_bd)


def _kmid_body(agg_ref, a_ref, s_ref, wlin2_ref, wcat_ref, s_next_ref, xl_next_ref):
    a = a_ref[...]
    agg = jnp.concatenate([agg_ref[0], agg_ref[1]], axis=1)
    z = _zcat(agg, a)
    o = jnp.dot(z, wlin2_ref[...], preferred_element_type=jnp.float32) * FSCALE
    hcur = C_S * s_ref[...] + C_X * o
    hcur = hcur * jax.nn.sigmoid(hcur)
    z2 = _zcat(hcur, a)
    szl = jnp.dot(z2, wcat_ref[...], preferred_element_type=jnp.float32) * FSCALE
    s_next_ref[...] = szl[:, :64]
    xl_next_ref[0] = szl[:, 64:96]
    xl_next_ref[1] = szl[:, 96:128]


def _kmid(agg, a, s_cur, wlin2f, wcat):
    return pl.pallas_call(
        _kmid_body,
        grid=(NGRID,),
        in_specs=[
            pl.BlockSpec((2, NBLK, 32), lambda i: (0, i, 0)),
            pl.BlockSpec((NBLK, 16), lambda i: (i, 0)),
            pl.BlockSpec((NBLK, 64), lambda i: (i, 0)),
            pl.BlockSpec((1024, 64), lambda i: (0, 0)),
            pl.BlockSpec((1024, 128), lambda i: (0, 0)),
        ],
        out_specs=[
            pl.BlockSpec((NBLK, 64), lambda i: (i, 0)),
            pl.BlockSpec((2, NBLK, 32), lambda i: (0, i, 0)),
        ],
        out_shape=[
            jax.ShapeDtypeStruct((NA, 64), jnp.float32),
            jax.ShapeDtypeStruct((2, NA, 32), jnp.float32),
        ],
    )(agg, a, s_cur, wlin2f, wcat)


def _kfin_body(agg_ref, a_ref, s_ref, wlin2_ref, o_ref):
    i = pl.program_id(0)
    a = a_ref[...]
    agg = jnp.concatenate([agg_ref[0], agg_ref[1]], axis=1)
    z = _zcat(agg, a)
    o = jnp.dot(z, wlin2_ref[...], preferred_element_type=jnp.float32) * FSCALE
    hout = C_S * s_ref[:, 0:1] + C_X * o
    rid = i * NBLK + lax.broadcasted_iota(jnp.int32, (NBLK, 1), 0)
    hout = jnp.where(rid < N, hout, 0.0)
    part = (jnp.sum(hout) * (1.0 / np.sqrt(float(N)))).reshape(1, 1)

    @pl.when(i == 0)
    def _():
        o_ref[...] = jnp.zeros((1, 1), jnp.float32)

    o_ref[...] += part


def _kfin(agg, a, s_cur, wlin2f):
    return pl.pallas_call(
        _kfin_body,
        grid=(NGRID,),
        in_specs=[
            pl.BlockSpec((2, NBLK, 32), lambda i: (0, i, 0)),
            pl.BlockSpec((NBLK, 16), lambda i: (i, 0)),
            pl.BlockSpec((NBLK, 64), lambda i: (i, 0)),
            pl.BlockSpec((1024, 1), lambda i: (0, 0)),
        ],
        out_specs=pl.BlockSpec((1, 1), lambda i: (0, 0)),
        out_shape=jax.ShapeDtypeStruct((1, 1), jnp.float32),
    )(agg, a, s_cur, wlin2f)


def kernel(x, x_attr, pos, edge_src, edge_dst, batch, W_em, b_em, W_ema, b_ema,
           sc0, lin1_0, fc1_0, fc2_0, lin2_0,
           sc1, lin1_1, fc1_1, fc2_1, lin2_1,
           sc2, lin1_2, fc1_2, fc2_2, lin2_2):
    xp = jnp.pad(x, ((0, NA - N), (0, 0)))
    xap = jnp.pad(x_attr, ((0, NA - N), (0, 0)))
    pos16 = jnp.pad(pos, ((0, 0), (0, 13)))

    srcpad = jnp.pad(edge_src, (0, EP - E))
    dstpad = jnp.pad(edge_dst, (0, EP - E))
    src2d = srcpad.reshape(NROW, 128)
    dst2d = dstpad.reshape(NROW, 128)
    srcp2d = jnp.concatenate([srcpad, srcpad + NA]).reshape(2 * NROW, 128)

    def _wt(w):
        return w.transpose(1, 0, 2).reshape(1024, w.shape[2])

    wcat0 = jnp.concatenate([_wt(sc0), _wt(lin1_0)], axis=1)
    wcat1 = jnp.concatenate([_wt(sc1), _wt(lin1_1)], axis=1)
    wcat2 = jnp.concatenate([jnp.pad(_wt(sc2), ((0, 0), (0, 63))), _wt(lin1_2)], axis=1)
    fc1_cat = jnp.zeros((16, 304), jnp.float32)
    fc2_bd = jnp.zeros((304, 192), jnp.float32)
    for l, (f1, f2) in enumerate(((fc1_0, fc2_0), (fc1_1, fc2_1), (fc1_2, fc2_2))):
        fc1_cat = fc1_cat.at[:10, 100 * l:100 * l + 100].set(f1)
        fc2_bd = fc2_bd.at[100 * l:100 * l + 100, 64 * l:64 * l + 64].set(f2)

    s_e = _geom(pos16, src2d, dst2d)
    cw_all = _kedge(s_e.reshape(EP, 1), fc1_cat, fc2_bd)
    cw_flat = cw_all.reshape(6 * EP, 32)

    a, s0, xl0 = _k1(xp, xap, W_em, b_em[None, :], W_ema, b_ema[None, :], wcat0)

    agg0 = _gs[0](xl0.reshape(2 * NA, 32), cw_flat, srcp2d, dst2d).reshape(2, NA, 32)
    s1, xl1 = _kmid(agg0, a, s0, _wt(lin2_0), wcat1)

    agg1 = _gs[1](xl1.reshape(2 * NA, 32), cw_flat, srcp2d, dst2d).reshape(2, NA, 32)
    s2, xl2 = _kmid(agg1, a, s1, _wt(lin2_1), wcat2)

    agg2 = _gs[2](xl2.reshape(2 * NA, 32), cw_flat, srcp2d, dst2d).reshape(2, NA, 32)
    return _kfin(agg2, a, s2, _wt(lin2_2))


# speedup vs baseline: 2.5287x; 1.2255x over previous
import functools

import jax
import jax.numpy as jnp
import numpy as np
from jax import lax
from jax.experimental import pallas as pl
from jax.experimental.pallas import tpu as pltpu
from jax.experimental.pallas import tpu_sc as plsc

N = 50000
E = 800000
NB = 10
RN = 100
MAX_RADIUS = 5.0

NA = 50048
NBLK = 1472
NGRID = NA // NBLK

EP = 819200
CHUNK = 256
KROW = CHUNK // 128
NROW = EP // 128
GCHUNKS = EP // 32 // CHUNK
ZROWS = NA // 16
ZQ = ZROWS // CHUNK
ZTAIL = ZROWS - ZQ * CHUNK

GC = 128
GI = 25
NGRP = EP // 16 // GC // GI

EBLK = 4096
C_S = float(np.sin(np.pi / 8.0))
C_X = float(np.cos(np.pi / 8.0))
FSCALE = 1.0 / 32.0

_SC_PARAMS = pltpu.CompilerParams(use_tc_tiling_on_sc=False,
                                  needs_layout_passes=False)
_MESH = plsc.VectorSubcoreMesh(core_axis_name="c", subcore_axis_name="s")


def _geom_body(pos_hbm, src_hbm, dst_hbm, out_hbm, idx_s, idx_d, ps, pd, sbuf, sem):
    c = lax.axis_index("c")
    s = lax.axis_index("s")
    wid = s * 2 + c
    base_row = wid * (GCHUNKS * KROW)
    lanes = lax.iota(jnp.int32, 16)
    col0 = jnp.zeros((16,), jnp.int32)
    col1 = col0 + 1
    col2 = col0 + 2

    def chunk_body(g, carry):
        j0 = base_row + g * KROW
        pltpu.sync_copy(src_hbm.at[pl.ds(j0, KROW)], idx_s)
        pltpu.sync_copy(dst_hbm.at[pl.ds(j0, KROW)], idx_d)
        ds_ = pltpu.async_copy(pos_hbm.at[idx_s.at[0]], ps.at[pl.ds(0, 128)], sem)
        ds2 = pltpu.async_copy(pos_hbm.at[idx_s.at[1]], ps.at[pl.ds(128, 128)], sem)
        dd_ = pltpu.async_copy(pos_hbm.at[idx_d.at[0]], pd.at[pl.ds(0, 128)], sem)
        dd2 = pltpu.async_copy(pos_hbm.at[idx_d.at[1]], pd.at[pl.ds(128, 128)], sem)
        ds_.wait(); ds2.wait(); dd_.wait(); dd2.wait()
        for v in range(16):
            ri = lanes + (v * 16)
            dx = plsc.load_gather(ps, [ri, col0]) - plsc.load_gather(pd, [ri, col0])
            dy = plsc.load_gather(ps, [ri, col1]) - plsc.load_gather(pd, [ri, col1])
            dz = plsc.load_gather(ps, [ri, col2]) - plsc.load_gather(pd, [ri, col2])
            sbuf[pl.ds(v * 16, 16)] = dx * dx + dy * dy + dz * dz
        pltpu.sync_copy(sbuf, out_hbm.at[pl.ds(j0 * 128, CHUNK)])
        return carry

    lax.fori_loop(0, GCHUNKS, chunk_body, 0)


_geom = functools.partial(
    pl.kernel,
    mesh=_MESH,
    compiler_params=_SC_PARAMS,
    out_type=jax.ShapeDtypeStruct((EP,), jnp.float32),
    scratch_types=[
        pltpu.VMEM((KROW, 128), jnp.int32),
        pltpu.VMEM((KROW, 128), jnp.int32),
        pltpu.VMEM((CHUNK, 16), jnp.float32),
        pltpu.VMEM((CHUNK, 16), jnp.float32),
        pltpu.VMEM((CHUNK,), jnp.float32),
        pltpu.SemaphoreType.DMA,
    ],
)(_geom_body)


def _make_gs(layer):
    def _gs_body(xl_hbm, cw_hbm, srcp_hbm, dst_hbm, out_hbm,
                 idx_s, idx_d, rows, cwv, aggsh,
                 gsem0, gsem1, csem0, csem1, tsem0, tsem1):
        c = lax.axis_index("c")
        s = lax.axis_index("s")
        zero16 = jnp.zeros((16,), jnp.float32)
        gsem = (gsem0, gsem1)
        csem = (csem0, csem1)
        tsem = (tsem0, tsem1)

        def zb_body(r, carry):
            rows[0, r, pl.ds(0, 16)] = zero16
            rows[0, r, pl.ds(16, 16)] = zero16
            return carry

        lax.fori_loop(0, GC, zb_body, 0)
        for q in range(ZROWS // GC):
            pltpu.sync_copy(rows.at[0], aggsh.at[pl.ds(s * ZROWS + q * GC, GC)])
        pltpu.sync_copy(rows.at[0].at[pl.ds(0, ZROWS % GC)],
                        aggsh.at[pl.ds(s * ZROWS + (ZROWS // GC) * GC, ZROWS % GC)])
        plsc.subcore_barrier()

        row_base = s * (NGRP * GI)
        cw_base = (2 * layer + c) * EP

        def group_body(grp, carry):
            j0 = row_base + grp * GI
            pltpu.sync_copy(srcp_hbm.at[pl.ds(c * NROW + j0, GI)], idx_s)
            pltpu.sync_copy(dst_hbm.at[pl.ds(j0, GI)], idx_d)

            def issue(g, b):
                pltpu.async_copy(xl_hbm.at[idx_s.at[g]], rows.at[b], gsem[b])
                pltpu.async_copy(cw_hbm.at[pl.ds(cw_base + (j0 + g) * 128, GC)],
                                 cwv.at[b], csem[b])

            def wait_in(b):
                pltpu.make_async_copy(xl_hbm.at[idx_s.at[0]], rows.at[b], gsem[b]).wait()
                pltpu.make_async_copy(cw_hbm.at[pl.ds(cw_base, GC)], cwv.at[b], csem[b]).wait()

            def wait_sc(b):
                pltpu.make_async_copy(rows.at[b], aggsh.at[idx_d.at[0]], tsem[b]).wait()

            def mul(b):
                def mul_body(r, cc):
                    rows[b, r, pl.ds(0, 16)] = rows[b, r, pl.ds(0, 16)] * cwv[b, r, pl.ds(0, 16)]
                    rows[b, r, pl.ds(16, 16)] = rows[b, r, pl.ds(16, 16)] * cwv[b, r, pl.ds(16, 16)]
                    return cc
                lax.fori_loop(0, GC, mul_body, 0)

            def scat(g, b):
                pltpu.async_copy(rows.at[b], aggsh.at[idx_d.at[g]], tsem[b], add=True)

            issue(0, 0)
            issue(1, 1)
            wait_in(0)
            mul(0)
            scat(0, 0)

            def pair_body(j, cc):
                g1 = 2 * j + 1
                wait_sc(0)
                issue(g1 + 1, 0)
                wait_in(1)
                mul(1)
                scat(g1, 1)
                wait_sc(1)
                issue(g1 + 2, 1)
                wait_in(0)
                mul(0)
                scat(g1 + 1, 0)
                return cc

            lax.fori_loop(0, (GI - 3) // 2, pair_body, 0)
            wait_sc(0)
            issue(GI - 1, 0)
            wait_in(1)
            mul(1)
            scat(GI - 2, 1)
            wait_in(0)
            mul(0)
            scat(GI - 1, 0)
            wait_sc(1)
            wait_sc(0)
            return carry

        lax.fori_loop(0, NGRP, group_body, 0)
        plsc.subcore_barrier()
        pltpu.sync_copy(aggsh.at[pl.ds(s * ZROWS, ZROWS)],
                        out_hbm.at[pl.ds(c * NA + s * ZROWS, ZROWS)])

    return functools.partial(
        pl.kernel,
        mesh=_MESH,
        compiler_params=_SC_PARAMS,
        out_type=jax.ShapeDtypeStruct((2 * NA, 32), jnp.float32),
        scratch_types=[
            pltpu.VMEM((GI, 128), jnp.int32),
            pltpu.VMEM((GI, 128), jnp.int32),
            pltpu.VMEM((2, GC, 32), jnp.float32),
            pltpu.VMEM((2, GC, 32), jnp.float32),
            pltpu.VMEM_SHARED((NA, 32), jnp.float32),
            pltpu.SemaphoreType.DMA,
            pltpu.SemaphoreType.DMA,
            pltpu.SemaphoreType.DMA,
            pltpu.SemaphoreType.DMA,
            pltpu.SemaphoreType.DMA,
            pltpu.SemaphoreType.DMA,
        ],
    )(_gs_body)


_gs = [_make_gs(l) for l in range(3)]


def _zcat(xmat, a):
    return jnp.concatenate([xmat * a[:, j:j + 1] for j in range(16)], axis=1)


def _k1_body(x_ref, xa_ref, wem_ref, bem_ref, wema_ref, bema_ref, wcat_ref,
             a_ref, s_ref, xl_ref):
    h = jnp.dot(x_ref[...], wem_ref[...], preferred_element_type=jnp.float32) + bem_ref[...]
    a = jnp.dot(xa_ref[...], wema_ref[...], preferred_element_type=jnp.float32) + bema_ref[...]
    a_ref[...] = a
    z = _zcat(h, a)
    szl = jnp.dot(z, wcat_ref[...], preferred_element_type=jnp.float32) * FSCALE
    s_ref[...] = szl[:, :64]
    xl_ref[0] = szl[:, 64:96]
    xl_ref[1] = szl[:, 96:128]


def _k1(xp, xap, W_em, b_em, W_ema, b_ema, wcat):
    return pl.pallas_call(
        _k1_body,
        grid=(NGRID,),
        in_specs=[
            pl.BlockSpec((NBLK, 128), lambda i: (i, 0)),
            pl.BlockSpec((NBLK, 16), lambda i: (i, 0)),
            pl.BlockSpec((128, 64), lambda i: (0, 0)),
            pl.BlockSpec((1, 64), lambda i: (0, 0)),
            pl.BlockSpec((16, 16), lambda i: (0, 0)),
            pl.BlockSpec((1, 16), lambda i: (0, 0)),
            pl.BlockSpec((1024, 128), lambda i: (0, 0)),
        ],
        out_specs=[
            pl.BlockSpec((NBLK, 16), lambda i: (i, 0)),
            pl.BlockSpec((NBLK, 64), lambda i: (i, 0)),
            pl.BlockSpec((2, NBLK, 32), lambda i: (0, i, 0)),
        ],
        out_shape=[
            jax.ShapeDtypeStruct((NA, 16), jnp.float32),
            jax.ShapeDtypeStruct((NA, 64), jnp.float32),
            jax.ShapeDtypeStruct((2, NA, 32), jnp.float32),
        ],
    )(xp, xap, W_em, b_em, W_ema, b_ema, wcat)


def _kedge_body(se_ref, fc1_ref, fc2_ref, cw_ref):
    i = pl.program_id(0)
    s = se_ref[...]
    el = jnp.sqrt(s + 1e-12)
    u = 2.0 * (el * (1.0 / MAX_RADIUS) - 1.0)
    y = (1.0 - jnp.cos(np.pi * u)) * 0.5
    y = jnp.where(u > 0, 0.0, y)
    y = jnp.where(u < -1, 1.0, y)
    rid = i * EBLK + lax.broadcasted_iota(jnp.int32, (EBLK, 1), 0)
    y = jnp.where(rid < E, y, 0.0) * 0.25
    step = MAX_RADIUS / (NB - 1)
    centers = lax.broadcasted_iota(jnp.int32, (1, 16), 1).astype(jnp.float32) * step
    emb = jnp.exp(-((el - centers) * (1.0 / step)) ** 2) * (1.12 * np.sqrt(NB))
    g = jnp.dot(emb, fc1_ref[...], preferred_element_type=jnp.float32) * (1.0 / np.sqrt(NB))
    h = g * jax.nn.sigmoid(g)
    w = jnp.dot(h, fc2_ref[...], preferred_element_type=jnp.float32) * 0.1
    cw = y * w
    for l in range(3):
        cw_ref[l, 0] = cw[:, 64 * l:64 * l + 32]
        cw_ref[l, 1] = cw[:, 64 * l + 32:64 * l + 64]


def _kedge(se2, fc1_cat, fc2_bd):
    return pl.pallas_call(
        _kedge_body,
        grid=(EP // EBLK,),
        in_specs=[
            pl.BlockSpec((EBLK, 1), lambda i: (i, 0)),
            pl.BlockSpec((16, 304), lambda i: (0, 0)),
            pl.BlockSpec((304, 192), lambda i: (0, 0)),
        ],
        out_specs=pl.BlockSpec((3, 2, EBLK, 32), lambda i: (0, 0, i, 0)),
        out_shape=jax.ShapeDtypeStruct((3, 2, EP, 32), jnp.float32),
    )(se2, fc1_cat, fc2_bd)


def _kmid_body(agg_ref, a_ref, s_ref, wlin2_ref, wcat_ref, s_next_ref, xl_next_ref):
    a = a_ref[...]
    agg = jnp.concatenate([agg_ref[0], agg_ref[1]], axis=1)
    z = _zcat(agg, a)
    o = jnp.dot(z, wlin2_ref[...], preferred_element_type=jnp.float32) * FSCALE
    hcur = C_S * s_ref[...] + C_X * o
    hcur = hcur * jax.nn.sigmoid(hcur)
    z2 = _zcat(hcur, a)
    szl = jnp.dot(z2, wcat_ref[...], preferred_element_type=jnp.float32) * FSCALE
    s_next_ref[...] = szl[:, :64]
    xl_next_ref[0] = szl[:, 64:96]
    xl_next_ref[1] = szl[:, 96:128]


def _kmid(agg, a, s_cur, wlin2f, wcat):
    return pl.pallas_call(
        _kmid_body,
        grid=(NGRID,),
        in_specs=[
            pl.BlockSpec((2, NBLK, 32), lambda i: (0, i, 0)),
            pl.BlockSpec((NBLK, 16), lambda i: (i, 0)),
            pl.BlockSpec((NBLK, 64), lambda i: (i, 0)),
            pl.BlockSpec((1024, 64), lambda i: (0, 0)),
            pl.BlockSpec((1024, 128), lambda i: (0, 0)),
        ],
        out_specs=[
            pl.BlockSpec((NBLK, 64), lambda i: (i, 0)),
            pl.BlockSpec((2, NBLK, 32), lambda i: (0, i, 0)),
        ],
        out_shape=[
            jax.ShapeDtypeStruct((NA, 64), jnp.float32),
            jax.ShapeDtypeStruct((2, NA, 32), jnp.float32),
        ],
    )(agg, a, s_cur, wlin2f, wcat)


def _kfin_body(agg_ref, a_ref, s_ref, wlin2_ref, o_ref):
    i = pl.program_id(0)
    a = a_ref[...]
    agg = jnp.concatenate([agg_ref[0], agg_ref[1]], axis=1)
    z = _zcat(agg, a)
    o = jnp.dot(z, wlin2_ref[...], preferred_element_type=jnp.float32) * FSCALE
    hout = C_S * s_ref[:, 0:1] + C_X * o
    rid = i * NBLK + lax.broadcasted_iota(jnp.int32, (NBLK, 1), 0)
    hout = jnp.where(rid < N, hout, 0.0)
    part = (jnp.sum(hout) * (1.0 / np.sqrt(float(N)))).reshape(1, 1)

    @pl.when(i == 0)
    def _():
        o_ref[...] = jnp.zeros((1, 1), jnp.float32)

    o_ref[...] += part


def _kfin(agg, a, s_cur, wlin2f):
    return pl.pallas_call(
        _kfin_body,
        grid=(NGRID,),
        in_specs=[
            pl.BlockSpec((2, NBLK, 32), lambda i: (0, i, 0)),
            pl.BlockSpec((NBLK, 16), lambda i: (i, 0)),
            pl.BlockSpec((NBLK, 64), lambda i: (i, 0)),
            pl.BlockSpec((1024, 1), lambda i: (0, 0)),
        ],
        out_specs=pl.BlockSpec((1, 1), lambda i: (0, 0)),
        out_shape=jax.ShapeDtypeStruct((1, 1), jnp.float32),
    )(agg, a, s_cur, wlin2f)


def kernel(x, x_attr, pos, edge_src, edge_dst, batch, W_em, b_em, W_ema, b_ema,
           sc0, lin1_0, fc1_0, fc2_0, lin2_0,
           sc1, lin1_1, fc1_1, fc2_1, lin2_1,
           sc2, lin1_2, fc1_2, fc2_2, lin2_2):
    xp = jnp.pad(x, ((0, NA - N), (0, 0)))
    xap = jnp.pad(x_attr, ((0, NA - N), (0, 0)))
    pos16 = jnp.pad(pos, ((0, 0), (0, 13)))

    srcpad = jnp.pad(edge_src, (0, EP - E))
    dstpad = jnp.pad(edge_dst, (0, EP - E))
    src2d = srcpad.reshape(NROW, 128)
    dst2d = dstpad.reshape(NROW, 128)
    srcp2d = jnp.concatenate([srcpad, srcpad + NA]).reshape(2 * NROW, 128)

    def _wt(w):
        return w.transpose(1, 0, 2).reshape(1024, w.shape[2])

    wcat0 = jnp.concatenate([_wt(sc0), _wt(lin1_0)], axis=1)
    wcat1 = jnp.concatenate([_wt(sc1), _wt(lin1_1)], axis=1)
    wcat2 = jnp.concatenate([jnp.pad(_wt(sc2), ((0, 0), (0, 63))), _wt(lin1_2)], axis=1)
    fc1_cat = jnp.zeros((16, 304), jnp.float32)
    fc2_bd = jnp.zeros((304, 192), jnp.float32)
    for l, (f1, f2) in enumerate(((fc1_0, fc2_0), (fc1_1, fc2_1), (fc1_2, fc2_2))):
        fc1_cat = fc1_cat.at[:10, 100 * l:100 * l + 100].set(f1)
        fc2_bd = fc2_bd.at[100 * l:100 * l + 100, 64 * l:64 * l + 64].set(f2)

    s_e = _geom(pos16, src2d, dst2d)
    cw_all = _kedge(s_e.reshape(EP, 1), fc1_cat, fc2_bd)
    cw_flat = cw_all.reshape(6 * EP, 32)

    a, s0, xl0 = _k1(xp, xap, W_em, b_em[None, :], W_ema, b_ema[None, :], wcat0)

    agg0 = _gs[0](xl0.reshape(2 * NA, 32), cw_flat, srcp2d, dst2d).reshape(2, NA, 32)
    s1, xl1 = _kmid(agg0, a, s0, _wt(lin2_0), wcat1)

    agg1 = _gs[1](xl1.reshape(2 * NA, 32), cw_flat, srcp2d, dst2d).reshape(2, NA, 32)
    s2, xl2 = _kmid(agg1, a, s1, _wt(lin2_1), wcat2)

    agg2 = _gs[2](xl2.reshape(2 * NA, 32), cw_flat, srcp2d, dst2d).reshape(2, NA, 32)
    return _kfin(agg2, a, s2, _wt(lin2_2))


# speedup vs baseline: 2.6165x; 1.0347x over previous
import functools

import jax
import jax.numpy as jnp
import numpy as np
from jax import lax
from jax.experimental import pallas as pl
from jax.experimental.pallas import tpu as pltpu
from jax.experimental.pallas import tpu_sc as plsc

N = 50000
E = 800000
NB = 10
RN = 100
MAX_RADIUS = 5.0

NA = 50048
NBLK = 1472
NGRID = NA // NBLK

EP = 819200
CHUNK = 256
KROW = CHUNK // 128
NROW = EP // 128
GCHUNKS = EP // 32 // CHUNK
ZROWS = NA // 16
ZQ = ZROWS // CHUNK
ZTAIL = ZROWS - ZQ * CHUNK

GC = 128
GI = 25
NGRP = EP // 16 // GC // GI

EBLK = 4096
C_S = float(np.sin(np.pi / 8.0))
C_X = float(np.cos(np.pi / 8.0))
FSCALE = 1.0 / 32.0

_SC_PARAMS = pltpu.CompilerParams(use_tc_tiling_on_sc=False,
                                  needs_layout_passes=False)
_MESH = plsc.VectorSubcoreMesh(core_axis_name="c", subcore_axis_name="s")


GGRP = 8


def _geom_body(pos_hbm, src_hbm, dst_hbm, out_hbm,
               idx_s, idx_d, ps, pd, sbuf, sem0, sem1):
    c = lax.axis_index("c")
    s = lax.axis_index("s")
    wid = s * 2 + c
    base_row = wid * (GGRP * GI)
    lanes = lax.iota(jnp.int32, 16)
    col0 = jnp.zeros((16,), jnp.int32)
    col1 = col0 + 1
    col2 = col0 + 2
    sem = (sem0, sem1)

    def group_body(grp, carry):
        j0 = base_row + grp * GI
        pltpu.sync_copy(src_hbm.at[pl.ds(j0, GI)], idx_s)
        pltpu.sync_copy(dst_hbm.at[pl.ds(j0, GI)], idx_d)

        def issue(g, b):
            pltpu.async_copy(pos_hbm.at[idx_s.at[g]], ps.at[b], sem[b])
            pltpu.async_copy(pos_hbm.at[idx_d.at[g]], pd.at[b], sem[b])

        def wait_in(b):
            pltpu.make_async_copy(pos_hbm.at[idx_s.at[0]], ps.at[b], sem[b]).wait()
            pltpu.make_async_copy(pos_hbm.at[idx_d.at[0]], pd.at[b], sem[b]).wait()

        def compute(g, b):
            for v in range(8):
                ri = lanes + (v * 16)
                dx = plsc.load_gather(ps.at[b], [ri, col0]) - plsc.load_gather(pd.at[b], [ri, col0])
                dy = plsc.load_gather(ps.at[b], [ri, col1]) - plsc.load_gather(pd.at[b], [ri, col1])
                dz = plsc.load_gather(ps.at[b], [ri, col2]) - plsc.load_gather(pd.at[b], [ri, col2])
                sbuf[pl.ds(g * 128 + v * 16, 16)] = dx * dx + dy * dy + dz * dz

        issue(0, 0)
        issue(1, 1)

        def pair_body(j, cc):
            g = 2 * j
            wait_in(0)
            compute(g, 0)
            issue(g + 2, 0)
            wait_in(1)
            compute(g + 1, 1)
            issue(g + 3, 1)
            return cc

        lax.fori_loop(0, (GI - 3) // 2, pair_body, 0)
        wait_in(0)
        compute(GI - 3, 0)
        issue(GI - 1, 0)
        wait_in(1)
        compute(GI - 2, 1)
        wait_in(0)
        compute(GI - 1, 0)
        pltpu.sync_copy(sbuf, out_hbm.at[pl.ds(j0 * 128, GI * 128)])
        return carry

    lax.fori_loop(0, GGRP, group_body, 0)


_geom = functools.partial(
    pl.kernel,
    mesh=_MESH,
    compiler_params=_SC_PARAMS,
    out_type=jax.ShapeDtypeStruct((EP,), jnp.float32),
    scratch_types=[
        pltpu.VMEM((GI, 128), jnp.int32),
        pltpu.VMEM((GI, 128), jnp.int32),
        pltpu.VMEM((2, 128, 16), jnp.float32),
        pltpu.VMEM((2, 128, 16), jnp.float32),
        pltpu.VMEM((GI * 128,), jnp.float32),
        pltpu.SemaphoreType.DMA,
        pltpu.SemaphoreType.DMA,
    ],
)(_geom_body)


def _make_gs(layer):
    def _gs_body(xl_hbm, cw_hbm, srcp_hbm, dst_hbm, out_hbm,
                 idx_s, idx_d, rows, cwv, aggsh,
                 gsem0, gsem1, csem0, csem1, tsem0, tsem1):
        c = lax.axis_index("c")
        s = lax.axis_index("s")
        zero16 = jnp.zeros((16,), jnp.float32)
        gsem = (gsem0, gsem1)
        csem = (csem0, csem1)
        tsem = (tsem0, tsem1)

        def zb_body(r, carry):
            rows[0, r, pl.ds(0, 16)] = zero16
            rows[0, r, pl.ds(16, 16)] = zero16
            return carry

        lax.fori_loop(0, GC, zb_body, 0)
        for q in range(ZROWS // GC):
            pltpu.sync_copy(rows.at[0], aggsh.at[pl.ds(s * ZROWS + q * GC, GC)])
        pltpu.sync_copy(rows.at[0].at[pl.ds(0, ZROWS % GC)],
                        aggsh.at[pl.ds(s * ZROWS + (ZROWS // GC) * GC, ZROWS % GC)])
        plsc.subcore_barrier()

        row_base = s * (NGRP * GI)

        def group_body(grp, carry):
            j0 = row_base + grp * GI
            pltpu.sync_copy(srcp_hbm.at[pl.ds(c * NROW + j0, GI)], idx_s)
            pltpu.sync_copy(dst_hbm.at[pl.ds(j0, GI)], idx_d)

            def issue(g, b):
                pltpu.async_copy(xl_hbm.at[idx_s.at[g]], rows.at[b], gsem[b])
                pltpu.async_copy(cw_hbm.at[layer, c, pl.ds((j0 + g) * 128, GC)],
                                 cwv.at[b], csem[b])

            def wait_in(b):
                pltpu.make_async_copy(xl_hbm.at[idx_s.at[0]], rows.at[b], gsem[b]).wait()
                pltpu.make_async_copy(cw_hbm.at[layer, c, pl.ds(0, GC)], cwv.at[b], csem[b]).wait()

            def wait_sc(b):
                pltpu.make_async_copy(rows.at[b], aggsh.at[idx_d.at[0]], tsem[b]).wait()

            def mul(b):
                def mul_body(r, cc):
                    r4 = r * 4
                    for k in range(4):
                        rows[b, r4 + k, pl.ds(0, 16)] = (
                            rows[b, r4 + k, pl.ds(0, 16)] * cwv[b, r4 + k, pl.ds(0, 16)])
                        rows[b, r4 + k, pl.ds(16, 16)] = (
                            rows[b, r4 + k, pl.ds(16, 16)] * cwv[b, r4 + k, pl.ds(16, 16)])
                    return cc
                lax.fori_loop(0, GC // 4, mul_body, 0)

            def scat(g, b):
                pltpu.async_copy(rows.at[b], aggsh.at[idx_d.at[g]], tsem[b], add=True)

            issue(0, 0)
            issue(1, 1)
            wait_in(0)
            mul(0)
            scat(0, 0)

            def pair_body(j, cc):
                g1 = 2 * j + 1
                wait_sc(0)
                issue(g1 + 1, 0)
                wait_in(1)
                mul(1)
                scat(g1, 1)
                wait_sc(1)
                issue(g1 + 2, 1)
                wait_in(0)
                mul(0)
                scat(g1 + 1, 0)
                return cc

            lax.fori_loop(0, (GI - 3) // 2, pair_body, 0)
            wait_sc(0)
            issue(GI - 1, 0)
            wait_in(1)
            mul(1)
            scat(GI - 2, 1)
            wait_in(0)
            mul(0)
            scat(GI - 1, 0)
            wait_sc(1)
            wait_sc(0)
            return carry

        lax.fori_loop(0, NGRP, group_body, 0)
        plsc.subcore_barrier()
        pltpu.sync_copy(aggsh.at[pl.ds(s * ZROWS, ZROWS)],
                        out_hbm.at[pl.ds(c * NA + s * ZROWS, ZROWS)])

    return functools.partial(
        pl.kernel,
        mesh=_MESH,
        compiler_params=_SC_PARAMS,
        out_type=jax.ShapeDtypeStruct((2 * NA, 32), jnp.float32),
        scratch_types=[
            pltpu.VMEM((GI, 128), jnp.int32),
            pltpu.VMEM((GI, 128), jnp.int32),
            pltpu.VMEM((2, GC, 32), jnp.float32),
            pltpu.VMEM((2, GC, 32), jnp.float32),
            pltpu.VMEM_SHARED((NA, 32), jnp.float32),
            pltpu.SemaphoreType.DMA,
            pltpu.SemaphoreType.DMA,
            pltpu.SemaphoreType.DMA,
            pltpu.SemaphoreType.DMA,
            pltpu.SemaphoreType.DMA,
            pltpu.SemaphoreType.DMA,
        ],
    )(_gs_body)


_gs = [_make_gs(l) for l in range(3)]


def _zcat(xmat, a):
    return jnp.concatenate([xmat * a[:, j:j + 1] for j in range(16)], axis=1)


def _k1_body(x_ref, xa_ref, wem_ref, bem_ref, wema_ref, bema_ref, wcat_ref,
             a_ref, s_ref, xl_ref):
    h = jnp.dot(x_ref[...], wem_ref[...], preferred_element_type=jnp.float32) + bem_ref[...]
    a = jnp.dot(xa_ref[...], wema_ref[...], preferred_element_type=jnp.float32) + bema_ref[...]
    a_ref[...] = a
    z = _zcat(h, a)
    szl = jnp.dot(z, wcat_ref[...], preferred_element_type=jnp.float32) * FSCALE
    s_ref[...] = szl[:, :64]
    xl_ref[0] = szl[:, 64:96]
    xl_ref[1] = szl[:, 96:128]


def _k1(xp, xap, W_em, b_em, W_ema, b_ema, wcat):
    return pl.pallas_call(
        _k1_body,
        grid=(NGRID,),
        in_specs=[
            pl.BlockSpec((NBLK, 128), lambda i: (i, 0)),
            pl.BlockSpec((NBLK, 16), lambda i: (i, 0)),
            pl.BlockSpec((128, 64), lambda i: (0, 0)),
            pl.BlockSpec((1, 64), lambda i: (0, 0)),
            pl.BlockSpec((16, 16), lambda i: (0, 0)),
            pl.BlockSpec((1, 16), lambda i: (0, 0)),
            pl.BlockSpec((1024, 128), lambda i: (0, 0)),
        ],
        out_specs=[
            pl.BlockSpec((NBLK, 16), lambda i: (i, 0)),
            pl.BlockSpec((NBLK, 64), lambda i: (i, 0)),
            pl.BlockSpec((2, NBLK, 32), lambda i: (0, i, 0)),
        ],
        out_shape=[
            jax.ShapeDtypeStruct((NA, 16), jnp.float32),
            jax.ShapeDtypeStruct((NA, 64), jnp.float32),
            jax.ShapeDtypeStruct((2, NA, 32), jnp.float32),
        ],
    )(xp, xap, W_em, b_em, W_ema, b_ema, wcat)


def _kedge_body(se_ref, fc1_ref, fc2_ref, cw_ref):
    i = pl.program_id(0)
    s = se_ref[...]
    el = jnp.sqrt(s + 1e-12)
    u = 2.0 * (el * (1.0 / MAX_RADIUS) - 1.0)
    y = (1.0 - jnp.cos(np.pi * u)) * 0.5
    y = jnp.where(u > 0, 0.0, y)
    y = jnp.where(u < -1, 1.0, y)
    rid = i * EBLK + lax.broadcasted_iota(jnp.int32, (EBLK, 1), 0)
    y = jnp.where(rid < E, y, 0.0) * 0.25
    step = MAX_RADIUS / (NB - 1)
    centers = lax.broadcasted_iota(jnp.int32, (1, 16), 1).astype(jnp.float32) * step
    emb = jnp.exp(-((el - centers) * (1.0 / step)) ** 2) * (1.12 * np.sqrt(NB))
    g = jnp.dot(emb, fc1_ref[...], preferred_element_type=jnp.float32) * (1.0 / np.sqrt(NB))
    h = g * jax.nn.sigmoid(g)
    w = jnp.dot(h, fc2_ref[...], preferred_element_type=jnp.float32) * 0.1
    cw = y * w
    for l in range(3):
        cw_ref[l, 0] = cw[:, 64 * l:64 * l + 32]
        cw_ref[l, 1] = cw[:, 64 * l + 32:64 * l + 64]


def _kedge(se2, fc1_cat, fc2_bd):
    return pl.pallas_call(
        _kedge_body,
        grid=(EP // EBLK,),
        in_specs=[
            pl.BlockSpec((EBLK, 1), lambda i: (i, 0)),
            pl.BlockSpec((16, 304), lambda i: (0, 0)),
            pl.BlockSpec((304, 192), lambda i: (0, 0)),
        ],
        out_specs=pl.BlockSpec((3, 2, EBLK, 32), lambda i: (0, 0, i, 0)),
        out_shape=jax.ShapeDtypeStruct((3, 2, EP, 32), jnp.float32),
    )(se2, fc1_cat, fc2_bd)


def _kmid_body(agg_ref, a_ref, s_ref, wlin2_ref, wcat_ref, s_next_ref, xl_next_ref):
    a = a_ref[...]
    agg = jnp.concatenate([agg_ref[0], agg_ref[1]], axis=1)
    z = _zcat(agg, a)
    o = jnp.dot(z, wlin2_ref[...], preferred_element_type=jnp.float32) * FSCALE
    hcur = C_S * s_ref[...] + C_X * o
    hcur = hcur * jax.nn.sigmoid(hcur)
    z2 = _zcat(hcur, a)
    szl = jnp.dot(z2, wcat_ref[...], preferred_element_type=jnp.float32) * FSCALE
    s_next_ref[...] = szl[:, :64]
    xl_next_ref[0] = szl[:, 64:96]
    xl_next_ref[1] = szl[:, 96:128]


def _kmid(agg, a, s_cur, wlin2f, wcat):
    return pl.pallas_call(
        _kmid_body,
        grid=(NGRID,),
        in_specs=[
            pl.BlockSpec((2, NBLK, 32), lambda i: (0, i, 0)),
            pl.BlockSpec((NBLK, 16), lambda i: (i, 0)),
            pl.BlockSpec((NBLK, 64), lambda i: (i, 0)),
            pl.BlockSpec((1024, 64), lambda i: (0, 0)),
            pl.BlockSpec((1024, 128), lambda i: (0, 0)),
        ],
        out_specs=[
            pl.BlockSpec((NBLK, 64), lambda i: (i, 0)),
            pl.BlockSpec((2, NBLK, 32), lambda i: (0, i, 0)),
        ],
        out_shape=[
            jax.ShapeDtypeStruct((NA, 64), jnp.float32),
            jax.ShapeDtypeStruct((2, NA, 32), jnp.float32),
        ],
    )(agg, a, s_cur, wlin2f, wcat)


def _kfin_body(agg_ref, a_ref, s_ref, wlin2_ref, o_ref):
    i = pl.program_id(0)
    a = a_ref[...]
    agg = jnp.concatenate([agg_ref[0], agg_ref[1]], axis=1)
    z = _zcat(agg, a)
    o = jnp.dot(z, wlin2_ref[...], preferred_element_type=jnp.float32) * FSCALE
    hout = C_S * s_ref[:, 0:1] + C_X * o
    rid = i * NBLK + lax.broadcasted_iota(jnp.int32, (NBLK, 1), 0)
    hout = jnp.where(rid < N, hout, 0.0)
    part = (jnp.sum(hout) * (1.0 / np.sqrt(float(N)))).reshape(1, 1)

    @pl.when(i == 0)
    def _():
        o_ref[...] = jnp.zeros((1, 1), jnp.float32)

    o_ref[...] += part


def _kfin(agg, a, s_cur, wlin2f):
    return pl.pallas_call(
        _kfin_body,
        grid=(NGRID,),
        in_specs=[
            pl.BlockSpec((2, NBLK, 32), lambda i: (0, i, 0)),
            pl.BlockSpec((NBLK, 16), lambda i: (i, 0)),
            pl.BlockSpec((NBLK, 64), lambda i: (i, 0)),
            pl.BlockSpec((1024, 1), lambda i: (0, 0)),
        ],
        out_specs=pl.BlockSpec((1, 1), lambda i: (0, 0)),
        out_shape=jax.ShapeDtypeStruct((1, 1), jnp.float32),
    )(agg, a, s_cur, wlin2f)


def kernel(x, x_attr, pos, edge_src, edge_dst, batch, W_em, b_em, W_ema, b_ema,
           sc0, lin1_0, fc1_0, fc2_0, lin2_0,
           sc1, lin1_1, fc1_1, fc2_1, lin2_1,
           sc2, lin1_2, fc1_2, fc2_2, lin2_2):
    xp = jnp.pad(x, ((0, NA - N), (0, 0)))
    xap = jnp.pad(x_attr, ((0, NA - N), (0, 0)))
    pos16 = jnp.pad(pos, ((0, 0), (0, 13)))

    srcpad = jnp.pad(edge_src, (0, EP - E))
    dstpad = jnp.pad(edge_dst, (0, EP - E))
    src2d = srcpad.reshape(NROW, 128)
    dst2d = dstpad.reshape(NROW, 128)
    srcp2d = jnp.concatenate([srcpad, srcpad + NA]).reshape(2 * NROW, 128)

    def _wt(w):
        return w.transpose(1, 0, 2).reshape(1024, w.shape[2])

    wcat0 = jnp.concatenate([_wt(sc0), _wt(lin1_0)], axis=1)
    wcat1 = jnp.concatenate([_wt(sc1), _wt(lin1_1)], axis=1)
    wcat2 = jnp.concatenate([jnp.pad(_wt(sc2), ((0, 0), (0, 63))), _wt(lin1_2)], axis=1)
    fc1_cat = jnp.zeros((16, 304), jnp.float32)
    fc2_bd = jnp.zeros((304, 192), jnp.float32)
    for l, (f1, f2) in enumerate(((fc1_0, fc2_0), (fc1_1, fc2_1), (fc1_2, fc2_2))):
        fc1_cat = fc1_cat.at[:10, 100 * l:100 * l + 100].set(f1)
        fc2_bd = fc2_bd.at[100 * l:100 * l + 100, 64 * l:64 * l + 64].set(f2)

    s_e = _geom(pos16, src2d, dst2d)
    cw_all = _kedge(s_e.reshape(EP, 1), fc1_cat, fc2_bd)

    a, s0, xl0 = _k1(xp, xap, W_em, b_em[None, :], W_ema, b_ema[None, :], wcat0)

    agg0 = _gs[0](xl0.reshape(2 * NA, 32), cw_all, srcp2d, dst2d).reshape(2, NA, 32)
    s1, xl1 = _kmid(agg0, a, s0, _wt(lin2_0), wcat1)

    agg1 = _gs[1](xl1.reshape(2 * NA, 32), cw_all, srcp2d, dst2d).reshape(2, NA, 32)
    s2, xl2 = _kmid(agg1, a, s1, _wt(lin2_1), wcat2)

    agg2 = _gs[2](xl2.reshape(2 * NA, 32), cw_all, srcp2d, dst2d).reshape(2, NA, 32)
    return _kfin(agg2, a, s2, _wt(lin2_2))


# speedup vs baseline: 2.9731x; 1.1363x over previous
import functools

import jax
import jax.numpy as jnp
import numpy as np
from jax import lax
from jax.experimental import pallas as pl
from jax.experimental.pallas import tpu as pltpu
from jax.experimental.pallas import tpu_sc as plsc

N = 50000
E = 800000
NB = 10
RN = 100
MAX_RADIUS = 5.0

NA = 50048
NBLK = 1472
NGRID = NA // NBLK

EP = 819200
CHUNK = 256
KROW = CHUNK // 128
NROW = EP // 128
GCHUNKS = EP // 32 // CHUNK
ZROWS = NA // 16
ZQ = ZROWS // CHUNK
ZTAIL = ZROWS - ZQ * CHUNK

GC = 128
GI = 25
NGRP = EP // 16 // GC // GI

EBLK = 4096
C_S = float(np.sin(np.pi / 8.0))
C_X = float(np.cos(np.pi / 8.0))
FSCALE = 1.0 / 32.0

_SC_PARAMS = pltpu.CompilerParams(use_tc_tiling_on_sc=False,
                                  needs_layout_passes=False)
_MESH = plsc.VectorSubcoreMesh(core_axis_name="c", subcore_axis_name="s")


GGRP = 8


def _geom_body(pos_hbm, src_hbm, dst_hbm, out_hbm,
               idx_s, idx_d, ps, pd, sbuf, sem0, sem1):
    c = lax.axis_index("c")
    s = lax.axis_index("s")
    wid = s * 2 + c
    base_row = wid * (GGRP * GI)
    lanes = lax.iota(jnp.int32, 16)
    col0 = jnp.zeros((16,), jnp.int32)
    col1 = col0 + 1
    col2 = col0 + 2
    sem = (sem0, sem1)

    def group_body(grp, carry):
        j0 = base_row + grp * GI
        pltpu.sync_copy(src_hbm.at[pl.ds(j0, GI)], idx_s)
        pltpu.sync_copy(dst_hbm.at[pl.ds(j0, GI)], idx_d)

        def issue(g, b):
            pltpu.async_copy(pos_hbm.at[idx_s.at[g]], ps.at[b], sem[b])
            pltpu.async_copy(pos_hbm.at[idx_d.at[g]], pd.at[b], sem[b])

        def wait_in(b):
            pltpu.make_async_copy(pos_hbm.at[idx_s.at[0]], ps.at[b], sem[b]).wait()
            pltpu.make_async_copy(pos_hbm.at[idx_d.at[0]], pd.at[b], sem[b]).wait()

        def compute(g, b):
            for v in range(8):
                ri = lanes + (v * 16)
                dx = plsc.load_gather(ps.at[b], [ri, col0]) - plsc.load_gather(pd.at[b], [ri, col0])
                dy = plsc.load_gather(ps.at[b], [ri, col1]) - plsc.load_gather(pd.at[b], [ri, col1])
                dz = plsc.load_gather(ps.at[b], [ri, col2]) - plsc.load_gather(pd.at[b], [ri, col2])
                sbuf[pl.ds(g * 128 + v * 16, 16)] = dx * dx + dy * dy + dz * dz

        issue(0, 0)
        issue(1, 1)

        def pair_body(j, cc):
            g = 2 * j
            wait_in(0)
            compute(g, 0)
            issue(g + 2, 0)
            wait_in(1)
            compute(g + 1, 1)
            issue(g + 3, 1)
            return cc

        lax.fori_loop(0, (GI - 3) // 2, pair_body, 0)
        wait_in(0)
        compute(GI - 3, 0)
        issue(GI - 1, 0)
        wait_in(1)
        compute(GI - 2, 1)
        wait_in(0)
        compute(GI - 1, 0)
        pltpu.sync_copy(sbuf, out_hbm.at[pl.ds(j0 * 128, GI * 128)])
        return carry

    lax.fori_loop(0, GGRP, group_body, 0)


_geom = functools.partial(
    pl.kernel,
    mesh=_MESH,
    compiler_params=_SC_PARAMS,
    out_type=jax.ShapeDtypeStruct((EP,), jnp.float32),
    scratch_types=[
        pltpu.VMEM((GI, 128), jnp.int32),
        pltpu.VMEM((GI, 128), jnp.int32),
        pltpu.VMEM((2, 128, 16), jnp.float32),
        pltpu.VMEM((2, 128, 16), jnp.float32),
        pltpu.VMEM((GI * 128,), jnp.float32),
        pltpu.SemaphoreType.DMA,
        pltpu.SemaphoreType.DMA,
    ],
)(_geom_body)


def _make_gs(layer):
    def _gs_body(xl_hbm, cw_hbm, srcp_hbm, dst_hbm, out_hbm,
                 idx_s, idx_d, rows, cwv, aggsh,
                 gsem0, gsem1, csem0, csem1, tsem0, tsem1):
        c = lax.axis_index("c")
        s = lax.axis_index("s")
        zero16 = jnp.zeros((16,), jnp.float32)
        gsem = (gsem0, gsem1)
        csem = (csem0, csem1)
        tsem = (tsem0, tsem1)

        def zb_body(r, carry):
            rows[0, r, pl.ds(0, 16)] = zero16
            rows[0, r, pl.ds(16, 16)] = zero16
            return carry

        lax.fori_loop(0, GC, zb_body, 0)
        for q in range(ZROWS // GC):
            pltpu.sync_copy(rows.at[0], aggsh.at[pl.ds(s * ZROWS + q * GC, GC)])
        pltpu.sync_copy(rows.at[0].at[pl.ds(0, ZROWS % GC)],
                        aggsh.at[pl.ds(s * ZROWS + (ZROWS // GC) * GC, ZROWS % GC)])
        plsc.subcore_barrier()

        row_base = s * (NGRP * GI)

        def group_body(grp, carry):
            j0 = row_base + grp * GI
            pltpu.sync_copy(srcp_hbm.at[pl.ds(c * NROW + j0, GI)], idx_s)
            pltpu.sync_copy(dst_hbm.at[pl.ds(j0, GI)], idx_d)

            def issue(g, b):
                pltpu.async_copy(xl_hbm.at[idx_s.at[g]], rows.at[b], gsem[b])
                pltpu.async_copy(cw_hbm.at[layer, c, pl.ds(0, 32), pl.ds((j0 + g) * 128, GC)],
                                 cwv.at[b], csem[b])

            def wait_in(b):
                pltpu.make_async_copy(xl_hbm.at[idx_s.at[0]], rows.at[b], gsem[b]).wait()
                pltpu.make_async_copy(cw_hbm.at[layer, c, pl.ds(0, 32), pl.ds(0, GC)],
                                      cwv.at[b], csem[b]).wait()

            def wait_sc(b):
                pltpu.make_async_copy(rows.at[b], aggsh.at[idx_d.at[0]], tsem[b]).wait()

            lanes = lax.iota(jnp.int32, 16)
            zi = jnp.zeros((16,), jnp.int32)

            def mul(b):
                def mul_body(r, cc):
                    r4 = r * 4
                    for k in range(4):
                        rc = zi + (r4 + k)
                        clo = plsc.load_gather(cwv.at[b], [lanes, rc])
                        chi = plsc.load_gather(cwv.at[b], [lanes + 16, rc])
                        rows[b, r4 + k, pl.ds(0, 16)] = rows[b, r4 + k, pl.ds(0, 16)] * clo
                        rows[b, r4 + k, pl.ds(16, 16)] = rows[b, r4 + k, pl.ds(16, 16)] * chi
                    return cc
                lax.fori_loop(0, GC // 4, mul_body, 0)

            def scat(g, b):
                pltpu.async_copy(rows.at[b], aggsh.at[idx_d.at[g]], tsem[b], add=True)

            issue(0, 0)
            issue(1, 1)
            wait_in(0)
            mul(0)
            scat(0, 0)

            def pair_body(j, cc):
                g1 = 2 * j + 1
                wait_sc(0)
                issue(g1 + 1, 0)
                wait_in(1)
                mul(1)
                scat(g1, 1)
                wait_sc(1)
                issue(g1 + 2, 1)
                wait_in(0)
                mul(0)
                scat(g1 + 1, 0)
                return cc

            lax.fori_loop(0, (GI - 3) // 2, pair_body, 0)
            wait_sc(0)
            issue(GI - 1, 0)
            wait_in(1)
            mul(1)
            scat(GI - 2, 1)
            wait_in(0)
            mul(0)
            scat(GI - 1, 0)
            wait_sc(1)
            wait_sc(0)
            return carry

        lax.fori_loop(0, NGRP, group_body, 0)
        plsc.subcore_barrier()
        pltpu.sync_copy(aggsh.at[pl.ds(s * ZROWS, ZROWS)],
                        out_hbm.at[pl.ds(c * NA + s * ZROWS, ZROWS)])

    return functools.partial(
        pl.kernel,
        mesh=_MESH,
        compiler_params=_SC_PARAMS,
        out_type=jax.ShapeDtypeStruct((2 * NA, 32), jnp.float32),
        scratch_types=[
            pltpu.VMEM((GI, 128), jnp.int32),
            pltpu.VMEM((GI, 128), jnp.int32),
            pltpu.VMEM((2, GC, 32), jnp.float32),
            pltpu.VMEM((2, 32, GC), jnp.float32),
            pltpu.VMEM_SHARED((NA, 32), jnp.float32),
            pltpu.SemaphoreType.DMA,
            pltpu.SemaphoreType.DMA,
            pltpu.SemaphoreType.DMA,
            pltpu.SemaphoreType.DMA,
            pltpu.SemaphoreType.DMA,
            pltpu.SemaphoreType.DMA,
        ],
    )(_gs_body)


_gs = [_make_gs(l) for l in range(3)]


def _zcat(xmat, a):
    return jnp.concatenate([xmat * a[:, j:j + 1] for j in range(16)], axis=1)


def _k1_body(x_ref, xa_ref, wem_ref, bem_ref, wema_ref, bema_ref, wcat_ref,
             a_ref, s_ref, xl_ref):
    h = jnp.dot(x_ref[...], wem_ref[...], preferred_element_type=jnp.float32) + bem_ref[...]
    a = jnp.dot(xa_ref[...], wema_ref[...], preferred_element_type=jnp.float32) + bema_ref[...]
    a_ref[...] = a
    z = _zcat(h, a)
    szl = jnp.dot(z, wcat_ref[...], preferred_element_type=jnp.float32) * FSCALE
    s_ref[...] = szl[:, :64]
    xl_ref[0] = szl[:, 64:96]
    xl_ref[1] = szl[:, 96:128]


def _k1(xp, xap, W_em, b_em, W_ema, b_ema, wcat):
    return pl.pallas_call(
        _k1_body,
        grid=(NGRID,),
        in_specs=[
            pl.BlockSpec((NBLK, 128), lambda i: (i, 0)),
            pl.BlockSpec((NBLK, 16), lambda i: (i, 0)),
            pl.BlockSpec((128, 64), lambda i: (0, 0)),
            pl.BlockSpec((1, 64), lambda i: (0, 0)),
            pl.BlockSpec((16, 16), lambda i: (0, 0)),
            pl.BlockSpec((1, 16), lambda i: (0, 0)),
            pl.BlockSpec((1024, 128), lambda i: (0, 0)),
        ],
        out_specs=[
            pl.BlockSpec((NBLK, 16), lambda i: (i, 0)),
            pl.BlockSpec((NBLK, 64), lambda i: (i, 0)),
            pl.BlockSpec((2, NBLK, 32), lambda i: (0, i, 0)),
        ],
        out_shape=[
            jax.ShapeDtypeStruct((NA, 16), jnp.float32),
            jax.ShapeDtypeStruct((NA, 64), jnp.float32),
            jax.ShapeDtypeStruct((2, NA, 32), jnp.float32),
        ],
    )(xp, xap, W_em, b_em, W_ema, b_ema, wcat)


def _kedge_body(se_ref, fc1t_ref, fc2t_ref, cw_ref):
    i = pl.program_id(0)
    s = se_ref[...]
    el = jnp.sqrt(s + 1e-12)
    u = 2.0 * (el * (1.0 / MAX_RADIUS) - 1.0)
    y = (1.0 - jnp.cos(np.pi * u)) * 0.5
    y = jnp.where(u > 0, 0.0, y)
    y = jnp.where(u < -1, 1.0, y)
    rid = i * EBLK + lax.broadcasted_iota(jnp.int32, (1, EBLK), 1)
    y = jnp.where(rid < E, y, 0.0) * 0.25
    step = MAX_RADIUS / (NB - 1)
    centers = lax.broadcasted_iota(jnp.int32, (16, 1), 0).astype(jnp.float32) * step
    emb = jnp.exp(-((el - centers) * (1.0 / step)) ** 2) * (1.12 * np.sqrt(NB))
    g = jnp.dot(fc1t_ref[...], emb, preferred_element_type=jnp.float32) * (1.0 / np.sqrt(NB))
    h = g * jax.nn.sigmoid(g)
    w = jnp.dot(fc2t_ref[...], h, preferred_element_type=jnp.float32) * 0.1
    cw = y * w
    for l in range(3):
        cw_ref[l, 0] = cw[64 * l:64 * l + 32, :]
        cw_ref[l, 1] = cw[64 * l + 32:64 * l + 64, :]


def _kedge(se2, fc1_t, fc2_t):
    return pl.pallas_call(
        _kedge_body,
        grid=(EP // EBLK,),
        in_specs=[
            pl.BlockSpec((1, EBLK), lambda i: (0, i)),
            pl.BlockSpec((304, 16), lambda i: (0, 0)),
            pl.BlockSpec((192, 304), lambda i: (0, 0)),
        ],
        out_specs=pl.BlockSpec((3, 2, 32, EBLK), lambda i: (0, 0, 0, i)),
        out_shape=jax.ShapeDtypeStruct((3, 2, 32, EP), jnp.float32),
    )(se2, fc1_t, fc2_t)


def _kmid_body(agg_ref, a_ref, s_ref, wlin2_ref, wcat_ref, s_next_ref, xl_next_ref):
    a = a_ref[...]
    agg = jnp.concatenate([agg_ref[0], agg_ref[1]], axis=1)
    z = _zcat(agg, a)
    o = jnp.dot(z, wlin2_ref[...], preferred_element_type=jnp.float32) * FSCALE
    hcur = C_S * s_ref[...] + C_X * o
    hcur = hcur * jax.nn.sigmoid(hcur)
    z2 = _zcat(hcur, a)
    szl = jnp.dot(z2, wcat_ref[...], preferred_element_type=jnp.float32) * FSCALE
    s_next_ref[...] = szl[:, :64]
    xl_next_ref[0] = szl[:, 64:96]
    xl_next_ref[1] = szl[:, 96:128]


def _kmid(agg, a, s_cur, wlin2f, wcat):
    return pl.pallas_call(
        _kmid_body,
        grid=(NGRID,),
        in_specs=[
            pl.BlockSpec((2, NBLK, 32), lambda i: (0, i, 0)),
            pl.BlockSpec((NBLK, 16), lambda i: (i, 0)),
            pl.BlockSpec((NBLK, 64), lambda i: (i, 0)),
            pl.BlockSpec((1024, 64), lambda i: (0, 0)),
            pl.BlockSpec((1024, 128), lambda i: (0, 0)),
        ],
        out_specs=[
            pl.BlockSpec((NBLK, 64), lambda i: (i, 0)),
            pl.BlockSpec((2, NBLK, 32), lambda i: (0, i, 0)),
        ],
        out_shape=[
            jax.ShapeDtypeStruct((NA, 64), jnp.float32),
            jax.ShapeDtypeStruct((2, NA, 32), jnp.float32),
        ],
    )(agg, a, s_cur, wlin2f, wcat)


def _kfin_body(agg_ref, a_ref, s_ref, wlin2_ref, o_ref):
    i = pl.program_id(0)
    a = a_ref[...]
    agg = jnp.concatenate([agg_ref[0], agg_ref[1]], axis=1)
    z = _zcat(agg, a)
    o = jnp.dot(z, wlin2_ref[...], preferred_element_type=jnp.float32) * FSCALE
    hout = C_S * s_ref[:, 0:1] + C_X * o
    rid = i * NBLK + lax.broadcasted_iota(jnp.int32, (NBLK, 1), 0)
    hout = jnp.where(rid < N, hout, 0.0)
    part = (jnp.sum(hout) * (1.0 / np.sqrt(float(N)))).reshape(1, 1)

    @pl.when(i == 0)
    def _():
        o_ref[...] = jnp.zeros((1, 1), jnp.float32)

    o_ref[...] += part


def _kfin(agg, a, s_cur, wlin2f):
    return pl.pallas_call(
        _kfin_body,
        grid=(NGRID,),
        in_specs=[
            pl.BlockSpec((2, NBLK, 32), lambda i: (0, i, 0)),
            pl.BlockSpec((NBLK, 16), lambda i: (i, 0)),
            pl.BlockSpec((NBLK, 64), lambda i: (i, 0)),
            pl.BlockSpec((1024, 1), lambda i: (0, 0)),
        ],
        out_specs=pl.BlockSpec((1, 1), lambda i: (0, 0)),
        out_shape=jax.ShapeDtypeStruct((1, 1), jnp.float32),
    )(agg, a, s_cur, wlin2f)


def kernel(x, x_attr, pos, edge_src, edge_dst, batch, W_em, b_em, W_ema, b_ema,
           sc0, lin1_0, fc1_0, fc2_0, lin2_0,
           sc1, lin1_1, fc1_1, fc2_1, lin2_1,
           sc2, lin1_2, fc1_2, fc2_2, lin2_2):
    xp = jnp.pad(x, ((0, NA - N), (0, 0)))
    xap = jnp.pad(x_attr, ((0, NA - N), (0, 0)))
    pos16 = jnp.pad(pos, ((0, 0), (0, 13)))

    srcpad = jnp.pad(edge_src, (0, EP - E))
    dstpad = jnp.pad(edge_dst, (0, EP - E))
    src2d = srcpad.reshape(NROW, 128)
    dst2d = dstpad.reshape(NROW, 128)
    srcp2d = jnp.concatenate([srcpad, srcpad + NA]).reshape(2 * NROW, 128)

    def _wt(w):
        return w.transpose(1, 0, 2).reshape(1024, w.shape[2])

    wcat0 = jnp.concatenate([_wt(sc0), _wt(lin1_0)], axis=1)
    wcat1 = jnp.concatenate([_wt(sc1), _wt(lin1_1)], axis=1)
    wcat2 = jnp.concatenate([jnp.pad(_wt(sc2), ((0, 0), (0, 63))), _wt(lin1_2)], axis=1)
    fc1_cat = jnp.zeros((16, 304), jnp.float32)
    fc2_bd = jnp.zeros((304, 192), jnp.float32)
    for l, (f1, f2) in enumerate(((fc1_0, fc2_0), (fc1_1, fc2_1), (fc1_2, fc2_2))):
        fc1_cat = fc1_cat.at[:10, 100 * l:100 * l + 100].set(f1)
        fc2_bd = fc2_bd.at[100 * l:100 * l + 100, 64 * l:64 * l + 64].set(f2)

    s_e = _geom(pos16, src2d, dst2d)
    cw_all = _kedge(s_e.reshape(1, EP), fc1_cat.T, fc2_bd.T)

    a, s0, xl0 = _k1(xp, xap, W_em, b_em[None, :], W_ema, b_ema[None, :], wcat0)

    agg0 = _gs[0](xl0.reshape(2 * NA, 32), cw_all, srcp2d, dst2d).reshape(2, NA, 32)
    s1, xl1 = _kmid(agg0, a, s0, _wt(lin2_0), wcat1)

    agg1 = _gs[1](xl1.reshape(2 * NA, 32), cw_all, srcp2d, dst2d).reshape(2, NA, 32)
    s2, xl2 = _kmid(agg1, a, s1, _wt(lin2_1), wcat2)

    agg2 = _gs[2](xl2.reshape(2 * NA, 32), cw_all, srcp2d, dst2d).reshape(2, NA, 32)
    return _kfin(agg2, a, s2, _wt(lin2_2))
